# Initial kernel scaffold; baseline (speedup 1.0000x reference)
import functools

import jax
import jax.numpy as jnp
from jax import lax
from jax.experimental import pallas as pl
from jax.experimental.pallas import tpu as pltpu
from jax.experimental.pallas import tpu_sc as plsc

N_NODES = 10000
N_EDGES = 320000
HID = 128
N_GRAPHS = 256
NP = 10240
HALF = 5000
HP = 5120
EP = N_EDGES + 128
DEGW = NP + N_GRAPHS
CFLAT = NP * HID
SFLAT = N_GRAPHS * HP

_mesh = plsc.VectorSubcoreMesh(core_axis_name="c", subcore_axis_name="s")
_f32 = jnp.float32
_i32 = jnp.int32


def _iota16():
    return lax.iota(_i32, 16)


def _zero_spmem_slice(zbuf, spmem, tile, words_per_tile, zchunk):
    @pl.loop(0, zchunk // 16)
    def _(i):
        zbuf[pl.ds(i * 16, 16)] = jnp.zeros((16,), _f32)

    @pl.loop(0, words_per_tile // zchunk)
    def _(i):
        pltpu.sync_copy(zbuf, spmem.at[pl.ds(tile * words_per_tile + i * zchunk, zchunk)])


def _scatter_chunks(nchunks, idx_buf, val_buf, spmem):
    @pl.loop(0, nchunks)
    def _(j):
        pltpu.sync_copy(val_buf.at[j], spmem.at[idx_buf.at[j]], add=True)


ECH_A = 79
NCH_A = 3
CH_A = ECH_A + NCH_A


@functools.partial(
    pl.kernel,
    out_type=jax.ShapeDtypeStruct((2, DEGW), _f32),
    mesh=_mesh,
    scratch_types=[
        pltpu.VMEM((NP,), _i32),
        pltpu.VMEM((ECH_A * 128,), _i32),
        pltpu.VMEM((CH_A, 128), _i32),
        pltpu.VMEM((CH_A, 128), _f32),
        pltpu.VMEM((656,), _f32),
        pltpu.VMEM_SHARED((DEGW,), _f32),
    ],
)
def _sc_deg(dst_hbm, batch_hbm, out_hbm, batch_t, dst_t, idx_b, val_b, zbuf, acc):
    sc = lax.axis_index("c")
    s = lax.axis_index("s")
    w = sc * 16 + s
    ebase = w * 10000
    nbase = w * 313

    pltpu.sync_copy(batch_hbm, batch_t)
    pltpu.sync_copy(dst_hbm.at[pl.ds(ebase, ECH_A * 128)], dst_t)

    _zero_spmem_slice(zbuf, acc, s, DEGW // 16, 656)
    plsc.subcore_barrier()

    @pl.loop(0, ECH_A)
    def _(j):
        for k in range(8):
            off = j * 128 + k * 16
            lane = off + _iota16()
            m = lane < 10000
            d16 = dst_t[pl.ds(off, 16)]
            idx_b[j, pl.ds(k * 16, 16)] = d16
            val_b[j, pl.ds(k * 16, 16)] = jnp.where(m, 1.0, 0.0).astype(_f32)

    @pl.loop(0, NCH_A)
    def _(jn):
        for k in range(8):
            off = jn * 128 + k * 16
            lane = off + _iota16()
            nv = nbase + lane
            m = (lane < 313) & (nv < N_NODES)
            b16 = plsc.load_gather(batch_t, [nv])
            idx_b[ECH_A + jn, pl.ds(k * 16, 16)] = NP + b16
            val_b[ECH_A + jn, pl.ds(k * 16, 16)] = jnp.where(m, 1.0, 0.0).astype(_f32)

    _scatter_chunks(CH_A, idx_b, val_b, acc)
    plsc.subcore_barrier()

    out_w = DEGW // 16
    pltpu.sync_copy(acc.at[pl.ds(s * out_w, out_w)], out_hbm.at[sc, pl.ds(s * out_w, out_w)])


ECH_B = 79
NCH_B = 3
CH_B = ECH_B + NCH_B


@functools.partial(
    pl.kernel,
    out_type=jax.ShapeDtypeStruct((2, CFLAT), _f32),
    mesh=_mesh,
    scratch_types=[
        pltpu.VMEM((NP,), _f32),
        pltpu.VMEM((NP,), _i32),
        pltpu.VMEM((ECH_B * 128,), _i32),
        pltpu.VMEM((ECH_B * 128,), _i32),
        pltpu.VMEM((CH_B, 128), _i32),
        pltpu.VMEM((CH_B, 128), _f32),
        pltpu.VMEM((4096,), _f32),
        pltpu.VMEM_SHARED((CFLAT,), _f32),
    ],
)
def _sc_cmat(src_hbm, dst_hbm, dinv_hbm, cls_hbm, out_hbm,
             dinv_t, cls_t, src_t, dst_t, idx_b, val_b, zbuf, acc):
    sc = lax.axis_index("c")
    s = lax.axis_index("s")
    w = sc * 16 + s
    ebase = w * 10000
    nbase = w * 313

    pltpu.sync_copy(dinv_hbm, dinv_t)
    pltpu.sync_copy(cls_hbm, cls_t)
    pltpu.sync_copy(src_hbm.at[pl.ds(ebase, ECH_B * 128)], src_t)
    pltpu.sync_copy(dst_hbm.at[pl.ds(ebase, ECH_B * 128)], dst_t)

    _zero_spmem_slice(zbuf, acc, s, CFLAT // 16, 4096)
    plsc.subcore_barrier()

    @pl.loop(0, ECH_B)
    def _(j):
        for k in range(8):
            off = j * 128 + k * 16
            lane = off + _iota16()
            m = lane < 10000
            s16 = src_t[pl.ds(off, 16)]
            d16 = dst_t[pl.ds(off, 16)]
            nrm = plsc.load_gather(dinv_t, [s16]) * plsc.load_gather(dinv_t, [d16])
            c16 = plsc.load_gather(cls_t, [s16])
            pnd = jnp.where(d16 >= HALF, d16 + 120, d16)
            idx_b[j, pl.ds(k * 16, 16)] = pnd * 128 + c16
            val_b[j, pl.ds(k * 16, 16)] = jnp.where(m, nrm, 0.0)

    @pl.loop(0, NCH_B)
    def _(jn):
        for k in range(8):
            off = jn * 128 + k * 16
            lane = off + _iota16()
            nv = nbase + lane
            m = (lane < 313) & (nv < N_NODES)
            dv = plsc.load_gather(dinv_t, [nv])
            c16 = plsc.load_gather(cls_t, [nv])
            pnn = jnp.where(nv >= HALF, nv + 120, nv)
            idx_b[ECH_B + jn, pl.ds(k * 16, 16)] = pnn * 128 + c16
            val_b[ECH_B + jn, pl.ds(k * 16, 16)] = jnp.where(m, dv * dv, 0.0)

    _scatter_chunks(CH_B, idx_b, val_b, acc)
    plsc.subcore_barrier()

    out_w = CFLAT // 16
    pltpu.sync_copy(acc.at[pl.ds(s * out_w, out_w)], out_hbm.at[sc, pl.ds(s * out_w, out_w)])


ECH_C = 157
NCH_C = 3
CH_C = ECH_C + NCH_C


@functools.partial(
    pl.kernel,
    out_type=jax.ShapeDtypeStruct((2, SFLAT), _f32),
    mesh=_mesh,
    scratch_types=[
        pltpu.VMEM((NP,), _f32),
        pltpu.VMEM((NP,), _i32),
        pltpu.VMEM((ECH_C * 128,), _i32),
        pltpu.VMEM((ECH_C * 128,), _i32),
        pltpu.VMEM((CH_C, 128), _i32),
        pltpu.VMEM((CH_C, 128), _f32),
        pltpu.VMEM((4096,), _f32),
        pltpu.VMEM_SHARED((SFLAT,), _f32),
    ],
)
def _sc_smat(src_hbm, dst_hbm, dinv_hbm, batch_hbm, out_hbm,
             dinv_t, batch_t, src_t, dst_t, idx_b, val_b, zbuf, acc):
    sc = lax.axis_index("c")
    s = lax.axis_index("s")
    ebase = s * 20000
    half_lo = sc * HALF
    nbase = half_lo + s * 313

    pltpu.sync_copy(dinv_hbm, dinv_t)
    pltpu.sync_copy(batch_hbm, batch_t)
    pltpu.sync_copy(src_hbm.at[pl.ds(ebase, ECH_C * 128)], src_t)
    pltpu.sync_copy(dst_hbm.at[pl.ds(ebase, ECH_C * 128)], dst_t)

    _zero_spmem_slice(zbuf, acc, s, SFLAT // 16, 4096)
    plsc.subcore_barrier()

    @pl.loop(0, ECH_C)
    def _(j):
        for k in range(8):
            off = j * 128 + k * 16
            lane = off + _iota16()
            s16 = src_t[pl.ds(off, 16)]
            d16 = dst_t[pl.ds(off, 16)]
            loc = s16 - half_lo
            m = (lane < 20000) & (loc >= 0) & (loc < HALF)
            nrm = plsc.load_gather(dinv_t, [s16]) * plsc.load_gather(dinv_t, [d16])
            b16 = plsc.load_gather(batch_t, [d16])
            locc = jnp.where(m, loc, s16 & 4095)
            idx_b[j, pl.ds(k * 16, 16)] = b16 * HP + locc
            val_b[j, pl.ds(k * 16, 16)] = jnp.where(m, nrm, 0.0)

    @pl.loop(0, NCH_C)
    def _(jn):
        for k in range(8):
            off = jn * 128 + k * 16
            lane = off + _iota16()
            nv = nbase + lane
            loc = nv - half_lo
            m = (lane < 313) & (loc < HALF)
            dv = plsc.load_gather(dinv_t, [nv])
            b16 = plsc.load_gather(batch_t, [nv])
            locc = jnp.where(m, loc, nv & 4095)
            idx_b[ECH_C + jn, pl.ds(k * 16, 16)] = b16 * HP + locc
            val_b[ECH_C + jn, pl.ds(k * 16, 16)] = jnp.where(m, dv * dv, 0.0)

    _scatter_chunks(CH_C, idx_b, val_b, acc)
    plsc.subcore_barrier()

    out_w = SFLAT // 16
    pltpu.sync_copy(acc.at[pl.ds(s * out_w, out_w)], out_hbm.at[sc, pl.ds(s * out_w, out_w)])


def _tc1_body(deg_ref, emb_ref, w1_ref, dinv_ref, t1_ref, cnt_ref):
    degsum = deg_ref[0] + deg_ref[1]
    dinv_ref[...] = lax.rsqrt(degsum[0:80] + 1.0)
    cnt_ref[...] = degsum[80:82]
    t1_ref[...] = jnp.dot(emb_ref[...], w1_ref[...], preferred_element_type=_f32)


def _tc1(deg3, embed_p, w1):
    return pl.pallas_call(
        _tc1_body,
        out_shape=[
            jax.ShapeDtypeStruct((80, 128), _f32),
            jax.ShapeDtypeStruct((128, 128), _f32),
            jax.ShapeDtypeStruct((2, 128), _f32),
        ],
    )(deg3, embed_p, w1)


def _tc2_body(c0_ref, c1_ref, t1_ref, b1_ref, w2_ref, hw_ref):
    a = c0_ref[...] + c1_ref[...]
    h = jnp.dot(a, t1_ref[...], preferred_element_type=_f32) + b1_ref[...]
    h = jnp.maximum(h, 0.0)
    hw_ref[...] = jnp.dot(h, w2_ref[...], preferred_element_type=_f32)


def _tc2(c0, c1, t1, b1r, w2):
    blk = 1024
    return pl.pallas_call(
        _tc2_body,
        grid=(NP // blk,),
        in_specs=[
            pl.BlockSpec((blk, 128), lambda i: (i, 0)),
            pl.BlockSpec((blk, 128), lambda i: (i, 0)),
            pl.BlockSpec((128, 128), lambda i: (0, 0)),
            pl.BlockSpec((1, 128), lambda i: (0, 0)),
            pl.BlockSpec((128, 128), lambda i: (0, 0)),
        ],
        out_specs=pl.BlockSpec((blk, 128), lambda i: (i, 0)),
        out_shape=jax.ShapeDtypeStruct((NP, 128), _f32),
    )(c0, c1, t1, b1r, w2)


def _tc3_body(s_ref, hw_ref, cnt_ref, b2_ref, out_ref):
    @pl.when((pl.program_id(0) == 0) & (pl.program_id(1) == 0))
    def _():
        out_ref[...] = cnt_ref[...] * b2_ref[...]

    out_ref[...] += jnp.dot(s_ref[0], hw_ref[...], preferred_element_type=_f32)


def _tc3(s3, hw, cntc, b2r):
    kblk = 256
    return pl.pallas_call(
        _tc3_body,
        grid=(2, HP // kblk),
        in_specs=[
            pl.BlockSpec((1, N_GRAPHS, kblk), lambda s, k: (s, 0, k)),
            pl.BlockSpec((kblk, 128), lambda s, k: (s * (HP // kblk) + k, 0)),
            pl.BlockSpec((N_GRAPHS, 1), lambda s, k: (0, 0)),
            pl.BlockSpec((1, 128), lambda s, k: (0, 0)),
        ],
        out_specs=pl.BlockSpec((N_GRAPHS, 128), lambda s, k: (0, 0)),
        out_shape=jax.ShapeDtypeStruct((N_GRAPHS, 128), _f32),
    )(s3, hw, cntc, b2r)


def kernel(x, edge_index, batch, embed_atom, W1, b1, W2, b2):
    cls_p = jnp.pad(x[:, 0].astype(_i32), (0, NP - N_NODODES if False else NP - N_NODES))
    batch_p = jnp.pad(batch.astype(_i32), (0, NP - N_NODES))
    src_p = jnp.pad(edge_index[0].astype(_i32), (0, EP - N_EDGES))
    dst_p = jnp.pad(edge_index[1].astype(_i32), (0, EP - N_EDGES))
    embed_p = jnp.pad(embed_atom.astype(_f32), ((0, 128 - embed_atom.shape[0]), (0, 0)))
    b1r = b1.reshape(1, HID).astype(_f32)
    b2r = b2.reshape(1, HID).astype(_f32)

    deg = _sc_deg(dst_p, batch_p)
    dinv2d, t1, cnt2 = _tc1(deg.reshape(2, 82, 128), embed_p, W1.astype(_f32))
    dinv = dinv2d.reshape(NP)

    cmat = _sc_cmat(src_p, dst_p, dinv, cls_p)
    smat = _sc_smat(src_p, dst_p, dinv, batch_p)

    c3 = cmat.reshape(2, NP, 128)
    hw = _tc2(c3[0], c3[1], t1, b1r, W2.astype(_f32))

    s3 = smat.reshape(2, N_GRAPHS, HP)
    return _tc3(s3, hw, cnt2.reshape(N_GRAPHS, 1), b2r)


# speedup vs baseline: 43.6612x; 43.6612x over previous
import dataclasses
import functools

import jax
import jax.numpy as jnp
from jax import lax
from jax.experimental import pallas as pl
from jax.experimental.pallas import tpu as pltpu
from jax.experimental.pallas import tpu_sc as plsc

N_NODES = 10000
N_EDGES = 320000
HID = 128
N_GRAPHS = 256
NP = 10240
HALF = 5000
HP = 5120
EP = N_EDGES + 2048
DEGW = NP + N_GRAPHS
CFLAT = NP * HID
SFLAT = N_GRAPHS * HP

_mesh = plsc.VectorSubcoreMesh(core_axis_name="c", subcore_axis_name="s")
_sc_params = pltpu.CompilerParams()
if "needs_layout_passes" in pltpu.CompilerParams.__dataclass_fields__:
    _sc_params = dataclasses.replace(_sc_params, needs_layout_passes=False)
_f32 = jnp.float32
_i32 = jnp.int32


def _iota16():
    return lax.iota(_i32, 16)


EBLK = 2048
ROWS = 16


def _zero_spmem_slice(zbuf, spmem, tile, words_per_tile, zchunk):
    @pl.loop(0, zchunk // 16)
    def _(i):
        zbuf[pl.ds(i * 16, 16)] = jnp.zeros((16,), _f32)

    @pl.loop(0, words_per_tile // zchunk)
    def _(i):
        pltpu.sync_copy(zbuf, spmem.at[pl.ds(tile * words_per_tile + i * zchunk, zchunk)])


def _scatter_rows(nrows, idx_buf, val_buf, spmem):
    @pl.loop(0, nrows)
    def _(r):
        pltpu.sync_copy(val_buf.at[r], spmem.at[idx_buf.at[r]], add=True)


def _copy_out(zbuf, acc, out_hbm, tile_words, spmem_base, hbm_base, chunk):
    @pl.loop(0, tile_words // chunk)
    def _(i):
        pltpu.sync_copy(acc.at[pl.ds(spmem_base + i * chunk, chunk)], zbuf.at[pl.ds(0, chunk)])
        pltpu.sync_copy(zbuf.at[pl.ds(0, chunk)], out_hbm.at[pl.ds(hbm_base + i * chunk, chunk)])


NBLK_A = 5


@functools.partial(
    pl.kernel,
    out_type=jax.ShapeDtypeStruct((2 * DEGW,), _f32),
    mesh=_mesh,
    compiler_params=_sc_params,
    scratch_types=[
        pltpu.VMEM((NP,), _i32),
        pltpu.VMEM((EBLK,), _i32),
        pltpu.VMEM((ROWS, 128), _i32),
        pltpu.VMEM((ROWS, 128), _f32),
        pltpu.VMEM((656,), _f32),
        pltpu.VMEM_SHARED((DEGW,), _f32),
    ],
)
def _sc_deg(dst_hbm, batch_hbm, out_hbm, batch_t, dst_t, idx_b, val_b, zbuf, acc):
    sc = lax.axis_index("c")
    s = lax.axis_index("s")
    w = sc * 16 + s
    ebase = w * 10000
    nbase = w * 313

    pltpu.sync_copy(batch_hbm, batch_t)
    _zero_spmem_slice(zbuf, acc, s, DEGW // 16, 656)
    plsc.subcore_barrier()

    @pl.loop(0, NBLK_A)
    def _(ib):
        pltpu.sync_copy(dst_hbm.at[pl.ds(ebase + ib * EBLK, EBLK)], dst_t)

        @pl.loop(0, ROWS)
        def _(r):
            for k in range(8):
                off = r * 128 + k * 16
                lane = ib * EBLK + off + _iota16()
                m = lane < 10000
                d16 = dst_t[pl.ds(off, 16)]
                idx_b[r, pl.ds(k * 16, 16)] = d16
                val_b[r, pl.ds(k * 16, 16)] = jnp.where(m, 1.0, 0.0).astype(_f32)

        _scatter_rows(ROWS, idx_b, val_b, acc)

    @pl.loop(0, 3)
    def _(r):
        for k in range(8):
            off = r * 128 + k * 16
            lane = off + _iota16()
            nv = nbase + lane
            m = (lane < 313) & (nv < N_NODES)
            b16 = plsc.load_gather(batch_t, [nv])
            idx_b[r, pl.ds(k * 16, 16)] = NP + b16
            val_b[r, pl.ds(k * 16, 16)] = jnp.where(m, 1.0, 0.0).astype(_f32)

    _scatter_rows(3, idx_b, val_b, acc)
    plsc.subcore_barrier()

    out_w = DEGW // 16
    pltpu.sync_copy(acc.at[pl.ds(s * out_w, out_w)], zbuf)
    pltpu.sync_copy(zbuf, out_hbm.at[pl.ds(sc * DEGW + s * out_w, out_w)])


NBLK_B = 5


@functools.partial(
    pl.kernel,
    out_type=jax.ShapeDtypeStruct((2 * CFLAT,), _f32),
    mesh=_mesh,
    compiler_params=_sc_params,
    scratch_types=[
        pltpu.VMEM((NP,), _f32),
        pltpu.VMEM((NP,), _i32),
        pltpu.VMEM((EBLK,), _i32),
        pltpu.VMEM((EBLK,), _i32),
        pltpu.VMEM((ROWS, 128), _i32),
        pltpu.VMEM((ROWS, 128), _f32),
        pltpu.VMEM((4096,), _f32),
        pltpu.VMEM_SHARED((CFLAT,), _f32),
    ],
)
def _sc_cmat(src_hbm, dst_hbm, dinv_hbm, cls_hbm, out_hbm,
             dinv_t, cls_t, src_t, dst_t, idx_b, val_b, zbuf, acc):
    sc = lax.axis_index("c")
    s = lax.axis_index("s")
    w = sc * 16 + s
    ebase = w * 10000
    nbase = w * 313

    pltpu.sync_copy(dinv_hbm, dinv_t)
    pltpu.sync_copy(cls_hbm, cls_t)
    _zero_spmem_slice(zbuf, acc, s, CFLAT // 16, 4096)
    plsc.subcore_barrier()

    @pl.loop(0, NBLK_B)
    def _(ib):
        pltpu.sync_copy(src_hbm.at[pl.ds(ebase + ib * EBLK, EBLK)], src_t)
        pltpu.sync_copy(dst_hbm.at[pl.ds(ebase + ib * EBLK, EBLK)], dst_t)

        @pl.loop(0, ROWS)
        def _(r):
            for k in range(8):
                off = r * 128 + k * 16
                lane = ib * EBLK + off + _iota16()
                m = lane < 10000
                s16 = src_t[pl.ds(off, 16)]
                d16 = dst_t[pl.ds(off, 16)]
                nrm = plsc.load_gather(dinv_t, [s16]) * plsc.load_gather(dinv_t, [d16])
                c16 = plsc.load_gather(cls_t, [s16])
                pnd = jnp.where(d16 >= HALF, d16 + 120, d16)
                idx_b[r, pl.ds(k * 16, 16)] = pnd * 128 + c16
                val_b[r, pl.ds(k * 16, 16)] = jnp.where(m, nrm, 0.0)

        _scatter_rows(ROWS, idx_b, val_b, acc)

    @pl.loop(0, 3)
    def _(r):
        for k in range(8):
            off = r * 128 + k * 16
            lane = off + _iota16()
            nv = nbase + lane
            m = (lane < 313) & (nv < N_NODES)
            dv = plsc.load_gather(dinv_t, [nv])
            c16 = plsc.load_gather(cls_t, [nv])
            pnn = jnp.where(nv >= HALF, nv + 120, nv)
            idx_b[r, pl.ds(k * 16, 16)] = pnn * 128 + c16
            val_b[r, pl.ds(k * 16, 16)] = jnp.where(m, dv * dv, 0.0)

    _scatter_rows(3, idx_b, val_b, acc)
    plsc.subcore_barrier()

    out_w = CFLAT // 16
    _copy_out(zbuf, acc, out_hbm, out_w, s * out_w, sc * CFLAT + s * out_w, 4096)


NBLK_C = 10


@functools.partial(
    pl.kernel,
    out_type=jax.ShapeDtypeStruct((2 * SFLAT,), _f32),
    mesh=_mesh,
    compiler_params=_sc_params,
    scratch_types=[
        pltpu.VMEM((NP,), _f32),
        pltpu.VMEM((NP,), _i32),
        pltpu.VMEM((EBLK,), _i32),
        pltpu.VMEM((EBLK,), _i32),
        pltpu.VMEM((ROWS, 128), _i32),
        pltpu.VMEM((ROWS, 128), _f32),
        pltpu.VMEM((4096,), _f32),
        pltpu.VMEM_SHARED((SFLAT,), _f32),
    ],
)
def _sc_smat(src_hbm, dst_hbm, dinv_hbm, batch_hbm, out_hbm,
             dinv_t, batch_t, src_t, dst_t, idx_b, val_b, zbuf, acc):
    sc = lax.axis_index("c")
    s = lax.axis_index("s")
    ebase = s * 20000
    half_lo = sc * HALF
    nbase = half_lo + s * 313

    pltpu.sync_copy(dinv_hbm, dinv_t)
    pltpu.sync_copy(batch_hbm, batch_t)
    _zero_spmem_slice(zbuf, acc, s, SFLAT // 16, 4096)
    plsc.subcore_barrier()

    @pl.loop(0, NBLK_C)
    def _(ib):
        pltpu.sync_copy(src_hbm.at[pl.ds(ebase + ib * EBLK, EBLK)], src_t)
        pltpu.sync_copy(dst_hbm.at[pl.ds(ebase + ib * EBLK, EBLK)], dst_t)

        @pl.loop(0, ROWS)
        def _(r):
            for k in range(8):
                off = r * 128 + k * 16
                lane = ib * EBLK + off + _iota16()
                s16 = src_t[pl.ds(off, 16)]
                d16 = dst_t[pl.ds(off, 16)]
                loc = s16 - half_lo
                m = (lane < 20000) & (loc >= 0) & (loc < HALF)
                nrm = plsc.load_gather(dinv_t, [s16]) * plsc.load_gather(dinv_t, [d16])
                b16 = plsc.load_gather(batch_t, [d16])
                locc = jnp.where(m, loc, s16 & 4095)
                idx_b[r, pl.ds(k * 16, 16)] = b16 * HP + locc
                val_b[r, pl.ds(k * 16, 16)] = jnp.where(m, nrm, 0.0)

        _scatter_rows(ROWS, idx_b, val_b, acc)

    @pl.loop(0, 3)
    def _(r):
        for k in range(8):
            off = r * 128 + k * 16
            lane = off + _iota16()
            nv = nbase + lane
            loc = nv - half_lo
            m = (lane < 313) & (loc < HALF)
            dv = plsc.load_gather(dinv_t, [nv])
            b16 = plsc.load_gather(batch_t, [nv])
            locc = jnp.where(m, loc, nv & 4095)
            idx_b[r, pl.ds(k * 16, 16)] = b16 * HP + locc
            val_b[r, pl.ds(k * 16, 16)] = jnp.where(m, dv * dv, 0.0)

    _scatter_rows(3, idx_b, val_b, acc)
    plsc.subcore_barrier()

    out_w = SFLAT // 16
    _copy_out(zbuf, acc, out_hbm, out_w, s * out_w, sc * SFLAT + s * out_w, 4096)


def _tc1_body(deg_ref, emb_ref, w1_ref, dinv_ref, t1_ref, cnt_ref):
    degsum = deg_ref[0] + deg_ref[1]
    dinv_ref[...] = lax.rsqrt(degsum[0:80] + 1.0)
    cnt_ref[...] = degsum[80:82]
    t1_ref[...] = jnp.dot(emb_ref[...], w1_ref[...], preferred_element_type=_f32)


def _tc1(deg3, embed_p, w1):
    return pl.pallas_call(
        _tc1_body,
        out_shape=[
            jax.ShapeDtypeStruct((80, 128), _f32),
            jax.ShapeDtypeStruct((128, 128), _f32),
            jax.ShapeDtypeStruct((2, 128), _f32),
        ],
    )(deg3, embed_p, w1)


def _tc2_body(c0_ref, c1_ref, t1_ref, b1_ref, w2_ref, hw_ref):
    a = c0_ref[...] + c1_ref[...]
    h = jnp.dot(a, t1_ref[...], preferred_element_type=_f32) + b1_ref[...]
    h = jnp.maximum(h, 0.0)
    hw_ref[...] = jnp.dot(h, w2_ref[...], preferred_element_type=_f32)


def _tc2(c0, c1, t1, b1r, w2):
    blk = 1024
    return pl.pallas_call(
        _tc2_body,
        grid=(NP // blk,),
        in_specs=[
            pl.BlockSpec((blk, 128), lambda i: (i, 0)),
            pl.BlockSpec((blk, 128), lambda i: (i, 0)),
            pl.BlockSpec((128, 128), lambda i: (0, 0)),
            pl.BlockSpec((1, 128), lambda i: (0, 0)),
            pl.BlockSpec((128, 128), lambda i: (0, 0)),
        ],
        out_specs=pl.BlockSpec((blk, 128), lambda i: (i, 0)),
        out_shape=jax.ShapeDtypeStruct((NP, 128), _f32),
    )(c0, c1, t1, b1r, w2)


def _tc3_body(s_ref, hw_ref, cnt_ref, b2_ref, out_ref):
    @pl.when((pl.program_id(0) == 0) & (pl.program_id(1) == 0))
    def _():
        out_ref[...] = cnt_ref[...] * b2_ref[...]

    out_ref[...] += jnp.dot(s_ref[0], hw_ref[...], preferred_element_type=_f32)


def _tc3(s3, hw, cntc, b2r):
    kblk = 256
    return pl.pallas_call(
        _tc3_body,
        grid=(2, HP // kblk),
        in_specs=[
            pl.BlockSpec((1, N_GRAPHS, kblk), lambda s, k: (s, 0, k)),
            pl.BlockSpec((kblk, 128), lambda s, k: (s * (HP // kblk) + k, 0)),
            pl.BlockSpec((N_GRAPHS, 1), lambda s, k: (0, 0)),
            pl.BlockSpec((1, 128), lambda s, k: (0, 0)),
        ],
        out_specs=pl.BlockSpec((N_GRAPHS, 128), lambda s, k: (0, 0)),
        out_shape=jax.ShapeDtypeStruct((N_GRAPHS, 128), _f32),
    )(s3, hw, cntc, b2r)


def kernel(x, edge_index, batch, embed_atom, W1, b1, W2, b2):
    cls_p = jnp.pad(x[:, 0].astype(_i32), (0, NP - N_NODES))
    batch_p = jnp.pad(batch.astype(_i32), (0, NP - N_NODES))
    src_p = jnp.pad(edge_index[0].astype(_i32), (0, EP - N_EDGES))
    dst_p = jnp.pad(edge_index[1].astype(_i32), (0, EP - N_EDGES))
    embed_p = jnp.pad(embed_atom.astype(_f32), ((0, 128 - embed_atom.shape[0]), (0, 0)))
    b1r = b1.reshape(1, HID).astype(_f32)
    b2r = b2.reshape(1, HID).astype(_f32)

    deg = _sc_deg(dst_p, batch_p)
    dinv2d, t1, cnt2 = _tc1(deg.reshape(2, 82, 128), embed_p, W1.astype(_f32))
    dinv = dinv2d.reshape(NP)

    cmat = _sc_cmat(src_p, dst_p, dinv, cls_p)
    smat = _sc_smat(src_p, dst_p, dinv, batch_p)

    c3 = cmat.reshape(2, NP, 128)
    hw = _tc2(c3[0], c3[1], t1, b1r, W2.astype(_f32))

    s3 = smat.reshape(2, N_GRAPHS, HP)
    return _tc3(s3, hw, cnt2.reshape(N_GRAPHS, 1), b2r)


# speedup vs baseline: 54.1413x; 1.2400x over previous
import dataclasses
import functools

import jax
import jax.numpy as jnp
from jax import lax
from jax.experimental import pallas as pl
from jax.experimental.pallas import tpu as pltpu
from jax.experimental.pallas import tpu_sc as plsc

N_NODES = 10000
N_EDGES = 320000
HID = 128
N_GRAPHS = 256
NP = 10240
HALF = 5000
HP = 5120
EP = N_EDGES + 4096
DEGW = NP + N_GRAPHS
CFLAT = NP * HID
SFLAT = N_GRAPHS * HP

_mesh = plsc.VectorSubcoreMesh(core_axis_name="c", subcore_axis_name="s")
_sc_params = pltpu.CompilerParams()
if "needs_layout_passes" in pltpu.CompilerParams.__dataclass_fields__:
    _sc_params = dataclasses.replace(_sc_params, needs_layout_passes=False)
_f32 = jnp.float32
_i32 = jnp.int32

EBLK = 2048
ROWS = 16


def _iota16():
    return lax.iota(_i32, 16)


def _rsqrt16(x):
    i = plsc.bitcast(x, _i32)
    i = 0x5F3759DF - lax.shift_right_logical(i, 1)
    y = plsc.bitcast(i, _f32)
    for _ in range(3):
        y = y * (1.5 - 0.5 * x * y * y)
    return y


def _fill_zbuf(zbuf, nwords):
    @pl.loop(0, nwords // 16)
    def _(i):
        zbuf[pl.ds(i * 16, 16)] = jnp.zeros((16,), _f32)


def _fire_zero(zbuf, acc, base, nwords, zchunk, sem):
    @pl.loop(0, nwords // zchunk)
    def _(i):
        pltpu.async_copy(zbuf.at[pl.ds(0, zchunk)], acc.at[pl.ds(base + i * zchunk, zchunk)], sem)


def _drain_zero(zbuf, acc, base, nwords, zchunk, sem):
    @pl.loop(0, nwords // zchunk)
    def _(i):
        pltpu.make_async_copy(zbuf.at[pl.ds(0, zchunk)], acc.at[pl.ds(base + i * zchunk, zchunk)], sem).wait()


def _fire_scatter(nrows, idx_b, val_b, acc, sem):
    @pl.loop(0, nrows)
    def _(r):
        pltpu.async_copy(val_b.at[r], acc.at[idx_b.at[r]], sem, add=True)


def _drain_scatter(nrows, idx_b, val_b, acc, sem):
    @pl.loop(0, nrows)
    def _(r):
        pltpu.make_async_copy(val_b.at[r], acc.at[idx_b.at[r]], sem).wait()


def _copy_out_async(bounce0, bounce1, acc, out_hbm, tile_words, spmem_base, hbm_base, chunk, sem):
    nch = tile_words // chunk

    @pl.loop(0, nch // 2)
    def _(jj):
        for half in range(2):
            buf = bounce0 if half == 0 else bounce1
            i = jj * 2 + half

            @pl.when(i >= 2)
            def _(i=i, buf=buf):
                pltpu.make_async_copy(
                    buf, out_hbm.at[pl.ds(hbm_base + (i - 2) * chunk, chunk)], sem).wait()

            pltpu.sync_copy(acc.at[pl.ds(spmem_base + i * chunk, chunk)], buf)
            pltpu.async_copy(buf, out_hbm.at[pl.ds(hbm_base + i * chunk, chunk)], sem)

    for half in range(2):
        buf = bounce0 if half == 0 else bounce1
        i = nch - 2 + half
        pltpu.make_async_copy(
            buf, out_hbm.at[pl.ds(hbm_base + i * chunk, chunk)], sem).wait()


NBLK_S = 10


@functools.partial(
    pl.kernel,
    out_type=(
        jax.ShapeDtypeStruct((N_GRAPHS,), _f32),
        jax.ShapeDtypeStruct((NP,), _f32),
        jax.ShapeDtypeStruct((2 * SFLAT,), _f32),
    ),
    mesh=_mesh,
    compiler_params=_sc_params,
    scratch_types=[
        pltpu.VMEM((NP,), _f32),
        pltpu.VMEM((NP,), _i32),
        pltpu.VMEM((EBLK,), _i32),
        pltpu.VMEM((EBLK,), _i32),
        pltpu.VMEM((EBLK,), _i32),
        pltpu.VMEM((EBLK,), _i32),
        pltpu.VMEM((ROWS, 128), _i32),
        pltpu.VMEM((ROWS, 128), _f32),
        pltpu.VMEM((4096,), _f32),
        pltpu.VMEM((4096,), _f32),
        pltpu.VMEM_SHARED((SFLAT + DEGW,), _f32),
        pltpu.SemaphoreType.DMA,
        pltpu.SemaphoreType.DMA,
        pltpu.SemaphoreType.DMA,
        pltpu.SemaphoreType.DMA,
    ],
)
def _sc_smat(src_hbm, dst_hbm, batch_hbm, cnt_hbm, dinv_hbm, out_hbm,
             dinv_t, batch_t, src_t0, src_t1, dst_t0, dst_t1, idx_b, val_b,
             bounce0, bounce1, acc,
             sem_z, sem_e, sem_s, sem_o):
    sc = lax.axis_index("c")
    s = lax.axis_index("s")
    ebase = s * 20000
    half_lo = sc * HALF

    _fill_zbuf(bounce0, 4096)
    _fire_zero(bounce0, acc, s * (SFLAT // 16), SFLAT // 16, 4096, sem_z)
    _fire_zero(bounce0, acc, SFLAT + s * (DEGW // 16), DEGW // 16, 656, sem_z)
    pltpu.sync_copy(batch_hbm, batch_t)
    pltpu.async_copy(dst_hbm.at[pl.ds(ebase, EBLK)], dst_t0, sem_e)
    _drain_zero(bounce0, acc, s * (SFLAT // 16), SFLAT // 16, 4096, sem_z)
    _drain_zero(bounce0, acc, SFLAT + s * (DEGW // 16), DEGW // 16, 656, sem_z)
    plsc.subcore_barrier()

    @pl.loop(0, NBLK_S // 2)
    def _(jj):
        for half in range(2):
            ib = jj * 2 + half
            cur_d = dst_t0 if half == 0 else dst_t1
            nxt_d = dst_t1 if half == 0 else dst_t0
            pltpu.make_async_copy(dst_hbm.at[pl.ds(ebase + ib * EBLK, EBLK)], cur_d, sem_e).wait()

            @pl.when(ib + 1 < NBLK_S)
            def _(ib=ib, nxt_d=nxt_d):
                pltpu.async_copy(dst_hbm.at[pl.ds(ebase + (ib + 1) * EBLK, EBLK)], nxt_d, sem_e)

            @pl.loop(0, ROWS)
            def _(r, ib=ib, cur_d=cur_d):
                for k in range(8):
                    off = r * 128 + k * 16
                    lane = ib * EBLK + off + _iota16()
                    m = lane < 20000
                    d16 = cur_d[pl.ds(off, 16)]
                    idx_b[r, pl.ds(k * 16, 16)] = SFLAT + d16
                    val_b[r, pl.ds(k * 16, 16)] = jnp.where(m, 1.0, 0.0).astype(_f32)

            _fire_scatter(ROWS, idx_b, val_b, acc, sem_s)
            _drain_scatter(ROWS, idx_b, val_b, acc, sem_s)

    @pl.loop(0, 5)
    def _(r):
        for k in range(8):
            off = r * 128 + k * 16
            nv = s * 640 + off + _iota16()
            m = nv < N_NODES
            b16 = plsc.load_gather(batch_t, [nv])
            idx_b[r, pl.ds(k * 16, 16)] = SFLAT + NP + b16
            val_b[r, pl.ds(k * 16, 16)] = jnp.where(m, 1.0, 0.0).astype(_f32)

    _fire_scatter(5, idx_b, val_b, acc, sem_s)
    _drain_scatter(5, idx_b, val_b, acc, sem_s)
    plsc.subcore_barrier()

    pltpu.async_copy(src_hbm.at[pl.ds(ebase, EBLK)], src_t0, sem_e)
    pltpu.async_copy(dst_hbm.at[pl.ds(ebase, EBLK)], dst_t0, sem_e)
    pltpu.sync_copy(acc.at[pl.ds(SFLAT, NP)], dinv_t)

    @pl.loop(0, NP // 16)
    def _(i):
        x = dinv_t[pl.ds(i * 16, 16)] + 1.0
        dinv_t[pl.ds(i * 16, 16)] = _rsqrt16(x)

    @pl.when(sc == 0)
    def _():
        pltpu.sync_copy(dinv_t.at[pl.ds(s * 640, 640)], dinv_hbm.at[pl.ds(s * 640, 640)])

        @pl.when(s == 0)
        def _():
            pltpu.sync_copy(acc.at[pl.ds(SFLAT + NP, N_GRAPHS)], bounce1.at[pl.ds(0, N_GRAPHS)])
            pltpu.sync_copy(bounce1.at[pl.ds(0, N_GRAPHS)], cnt_hbm)

    @pl.loop(0, NBLK_S // 2)
    def _(jj):
        for half in range(2):
            ib = jj * 2 + half
            cur_s = src_t0 if half == 0 else src_t1
            cur_d = dst_t0 if half == 0 else dst_t1
            nxt_s = src_t1 if half == 0 else src_t0
            nxt_d = dst_t1 if half == 0 else dst_t0
            pltpu.make_async_copy(src_hbm.at[pl.ds(ebase + ib * EBLK, EBLK)], cur_s, sem_e).wait()
            pltpu.make_async_copy(dst_hbm.at[pl.ds(ebase + ib * EBLK, EBLK)], cur_d, sem_e).wait()

            @pl.when(ib + 1 < NBLK_S)
            def _(ib=ib, nxt_s=nxt_s, nxt_d=nxt_d):
                pltpu.async_copy(src_hbm.at[pl.ds(ebase + (ib + 1) * EBLK, EBLK)], nxt_s, sem_e)
                pltpu.async_copy(dst_hbm.at[pl.ds(ebase + (ib + 1) * EBLK, EBLK)], nxt_d, sem_e)

            @pl.loop(0, ROWS)
            def _(r, ib=ib, cur_s=cur_s, cur_d=cur_d):
                for k in range(8):
                    off = r * 128 + k * 16
                    lane = ib * EBLK + off + _iota16()
                    s16 = cur_s[pl.ds(off, 16)]
                    d16 = cur_d[pl.ds(off, 16)]
                    loc = s16 - half_lo
                    m = (lane < 20000) & (loc >= 0) & (loc < HALF)
                    nrm = plsc.load_gather(dinv_t, [s16]) * plsc.load_gather(dinv_t, [d16])
                    b16 = plsc.load_gather(batch_t, [d16])
                    locc = jnp.where(m, loc, s16 & 4095)
                    idx_b[r, pl.ds(k * 16, 16)] = b16 * HP + locc
                    val_b[r, pl.ds(k * 16, 16)] = jnp.where(m, nrm, 0.0)

            _fire_scatter(ROWS, idx_b, val_b, acc, sem_s)
            _drain_scatter(ROWS, idx_b, val_b, acc, sem_s)

    nbase = half_lo + s * 313

    @pl.loop(0, 3)
    def _(r):
        for k in range(8):
            off = r * 128 + k * 16
            lane = off + _iota16()
            nv = nbase + lane
            loc = nv - half_lo
            m = (lane < 313) & (loc < HALF)
            dv = plsc.load_gather(dinv_t, [nv])
            b16 = plsc.load_gather(batch_t, [nv])
            locc = jnp.where(m, loc, nv & 4095)
            idx_b[r, pl.ds(k * 16, 16)] = b16 * HP + locc
            val_b[r, pl.ds(k * 16, 16)] = jnp.where(m, dv * dv, 0.0)

    _fire_scatter(3, idx_b, val_b, acc, sem_s)
    _drain_scatter(3, idx_b, val_b, acc, sem_s)
    plsc.subcore_barrier()

    out_w = SFLAT // 16
    _copy_out_async(bounce0, bounce1, acc, out_hbm, out_w, s * out_w, sc * SFLAT + s * out_w, 4096, sem_o)


NBLK_B = 6


@functools.partial(
    pl.kernel,
    out_type=jax.ShapeDtypeStruct((2 * CFLAT,), _f32),
    mesh=_mesh,
    compiler_params=_sc_params,
    scratch_types=[
        pltpu.VMEM((NP,), _f32),
        pltpu.VMEM((NP,), _i32),
        pltpu.VMEM((EBLK,), _i32),
        pltpu.VMEM((EBLK,), _i32),
        pltpu.VMEM((EBLK,), _i32),
        pltpu.VMEM((EBLK,), _i32),
        pltpu.VMEM((ROWS, 128), _i32),
        pltpu.VMEM((ROWS, 128), _f32),
        pltpu.VMEM((4096,), _f32),
        pltpu.VMEM((4096,), _f32),
        pltpu.VMEM_SHARED((CFLAT,), _f32),
        pltpu.SemaphoreType.DMA,
        pltpu.SemaphoreType.DMA,
        pltpu.SemaphoreType.DMA,
        pltpu.SemaphoreType.DMA,
    ],
)
def _sc_cmat(src_hbm, dst_hbm, dinv_hbm, cls_hbm, out_hbm,
             dinv_t, cls_t, src_t0, src_t1, dst_t0, dst_t1, idx_b, val_b,
             bounce0, bounce1, acc,
             sem_z, sem_e, sem_s, sem_o):
    sc = lax.axis_index("c")
    s = lax.axis_index("s")
    w = sc * 16 + s
    ebase = w * 10000
    nbase = w * 313

    _fill_zbuf(bounce0, 4096)
    _fire_zero(bounce0, acc, s * (CFLAT // 16), CFLAT // 16, 4096, sem_z)
    pltpu.sync_copy(dinv_hbm, dinv_t)
    pltpu.sync_copy(cls_hbm, cls_t)
    pltpu.async_copy(src_hbm.at[pl.ds(ebase, EBLK)], src_t0, sem_e)
    pltpu.async_copy(dst_hbm.at[pl.ds(ebase, EBLK)], dst_t0, sem_e)
    _drain_zero(bounce0, acc, s * (CFLAT // 16), CFLAT // 16, 4096, sem_z)
    plsc.subcore_barrier()

    @pl.loop(0, NBLK_B // 2)
    def _(jj):
        for half in range(2):
            ib = jj * 2 + half
            cur_s = src_t0 if half == 0 else src_t1
            cur_d = dst_t0 if half == 0 else dst_t1
            nxt_s = src_t1 if half == 0 else src_t0
            nxt_d = dst_t1 if half == 0 else dst_t0
            pltpu.make_async_copy(src_hbm.at[pl.ds(ebase + ib * EBLK, EBLK)], cur_s, sem_e).wait()
            pltpu.make_async_copy(dst_hbm.at[pl.ds(ebase + ib * EBLK, EBLK)], cur_d, sem_e).wait()

            @pl.when(ib + 1 < NBLK_B)
            def _(ib=ib, nxt_s=nxt_s, nxt_d=nxt_d):
                pltpu.async_copy(src_hbm.at[pl.ds(ebase + (ib + 1) * EBLK, EBLK)], nxt_s, sem_e)
                pltpu.async_copy(dst_hbm.at[pl.ds(ebase + (ib + 1) * EBLK, EBLK)], nxt_d, sem_e)

            @pl.loop(0, ROWS)
            def _(r, ib=ib, cur_s=cur_s, cur_d=cur_d):
                for k in range(8):
                    off = r * 128 + k * 16
                    lane = ib * EBLK + off + _iota16()
                    m = lane < 10000
                    s16 = cur_s[pl.ds(off, 16)]
                    d16 = cur_d[pl.ds(off, 16)]
                    nrm = plsc.load_gather(dinv_t, [s16]) * plsc.load_gather(dinv_t, [d16])
                    c16 = plsc.load_gather(cls_t, [s16])
                    pnd = jnp.where(d16 >= HALF, d16 + 120, d16)
                    idx_b[r, pl.ds(k * 16, 16)] = pnd * 128 + c16
                    val_b[r, pl.ds(k * 16, 16)] = jnp.where(m, nrm, 0.0)

            _fire_scatter(ROWS, idx_b, val_b, acc, sem_s)
            _drain_scatter(ROWS, idx_b, val_b, acc, sem_s)

    @pl.loop(0, 3)
    def _(r):
        for k in range(8):
            off = r * 128 + k * 16
            lane = off + _iota16()
            nv = nbase + lane
            m = (lane < 313) & (nv < N_NODES)
            dv = plsc.load_gather(dinv_t, [nv])
            c16 = plsc.load_gather(cls_t, [nv])
            pnn = jnp.where(nv >= HALF, nv + 120, nv)
            idx_b[r, pl.ds(k * 16, 16)] = pnn * 128 + c16
            val_b[r, pl.ds(k * 16, 16)] = jnp.where(m, dv * dv, 0.0)

    _fire_scatter(3, idx_b, val_b, acc, sem_s)
    _drain_scatter(3, idx_b, val_b, acc, sem_s)
    plsc.subcore_barrier()

    out_w = CFLAT // 16
    _copy_out_async(bounce0, bounce1, acc, out_hbm, out_w, s * out_w, sc * CFLAT + s * out_w, 4096, sem_o)


KBLK = 256


def _tc_body(c0_ref, c1_ref, emb_ref, w1_ref, b1_ref, w2_ref, s_ref, cnt_ref,
             b2_ref, out_ref, t1_ref):
    @pl.when((pl.program_id(0) == 0) & (pl.program_id(1) == 0))
    def _():
        t1_ref[...] = jnp.dot(emb_ref[...], w1_ref[...], preferred_element_type=_f32)
        out_ref[...] = cnt_ref[...] * b2_ref[...]

    a = c0_ref[...] + c1_ref[...]
    h = jnp.dot(a, t1_ref[...], preferred_element_type=_f32) + b1_ref[...]
    h = jnp.maximum(h, 0.0)
    hw = jnp.dot(h, w2_ref[...], preferred_element_type=_f32)
    out_ref[...] += jnp.dot(s_ref[0], hw, preferred_element_type=_f32)


def _tc_dense(c0, c1, embed_p, w1, b1r, w2, s3, cntc, b2r):
    nk = HP // KBLK
    return pl.pallas_call(
        _tc_body,
        grid=(2, nk),
        in_specs=[
            pl.BlockSpec((KBLK, 128), lambda s, k: (s * (HP // KBLK) + k, 0)),
            pl.BlockSpec((KBLK, 128), lambda s, k: (s * (HP // KBLK) + k, 0)),
            pl.BlockSpec((128, 128), lambda s, k: (0, 0)),
            pl.BlockSpec((128, 128), lambda s, k: (0, 0)),
            pl.BlockSpec((1, 128), lambda s, k: (0, 0)),
            pl.BlockSpec((128, 128), lambda s, k: (0, 0)),
            pl.BlockSpec((1, N_GRAPHS, KBLK), lambda s, k: (s, 0, k)),
            pl.BlockSpec((N_GRAPHS, 1), lambda s, k: (0, 0)),
            pl.BlockSpec((1, 128), lambda s, k: (0, 0)),
        ],
        out_specs=pl.BlockSpec((N_GRAPHS, 128), lambda s, k: (0, 0)),
        out_shape=jax.ShapeDtypeStruct((N_GRAPHS, 128), _f32),
        scratch_shapes=[pltpu.VMEM((128, 128), _f32)],
    )(c0, c1, embed_p, w1, b1r, w2, s3, cntc, b2r)


def kernel(x, edge_index, batch, embed_atom, W1, b1, W2, b2):
    cls_p = jnp.pad(x[:, 0].astype(_i32), (0, NP - N_NODES))
    batch_p = jnp.pad(batch.astype(_i32), (0, NP - N_NODES))
    src_p = jnp.pad(edge_index[0].astype(_i32), (0, EP - N_EDGES))
    dst_p = jnp.pad(edge_index[1].astype(_i32), (0, EP - N_EDGES))
    embed_p = jnp.pad(embed_atom.astype(_f32), ((0, 128 - embed_atom.shape[0]), (0, 0)))
    b1r = b1.reshape(1, HID).astype(_f32)
    b2r = b2.reshape(1, HID).astype(_f32)

    cnt, dinv, smat = _sc_smat(src_p, dst_p, batch_p)
    cmat = _sc_cmat(src_p, dst_p, dinv, cls_p)

    c3 = cmat.reshape(2, NP, 128)
    s3 = smat.reshape(2, N_GRAPHS, HP)
    return _tc_dense(c3[0], c3[1], embed_p, W1.astype(_f32), b1r,
                     W2.astype(_f32), s3, cnt.reshape(N_GRAPHS, 1), b2r)


# speedup vs baseline: 58.0245x; 1.0717x over previous
import dataclasses
import functools

import jax
import jax.numpy as jnp
from jax import lax
from jax.experimental import pallas as pl
from jax.experimental.pallas import tpu as pltpu
from jax.experimental.pallas import tpu_sc as plsc

N_NODES = 10000
N_EDGES = 320000
HID = 128
N_GRAPHS = 256
NP = 10240
HALF = 5000
HP = 5120
EP = N_EDGES + 4096
DEGW = NP + N_GRAPHS
CFLAT = NP * HID
SFLAT = N_GRAPHS * HP

_mesh = plsc.VectorSubcoreMesh(core_axis_name="c", subcore_axis_name="s")
_sc_params = pltpu.CompilerParams()
if "needs_layout_passes" in pltpu.CompilerParams.__dataclass_fields__:
    _sc_params = dataclasses.replace(_sc_params, needs_layout_passes=False)
_f32 = jnp.float32
_i32 = jnp.int32

EBLK = 2048
ROWS = 16


def _iota16():
    return lax.iota(_i32, 16)


def _rsqrt16(x):
    i = plsc.bitcast(x, _i32)
    i = 0x5F3759DF - lax.shift_right_logical(i, 1)
    y = plsc.bitcast(i, _f32)
    for _ in range(3):
        y = y * (1.5 - 0.5 * x * y * y)
    return y


def _fill_zbuf(zbuf, nwords):
    @pl.loop(0, nwords // 16)
    def _(i):
        zbuf[pl.ds(i * 16, 16)] = jnp.zeros((16,), _f32)


def _fire_zero(zbuf, acc, base, nwords, zchunk, sem):
    @pl.loop(0, nwords // zchunk)
    def _(i):
        pltpu.async_copy(zbuf.at[pl.ds(0, zchunk)], acc.at[pl.ds(base + i * zchunk, zchunk)], sem)


def _drain_zero(zbuf, acc, base, nwords, zchunk, sem):
    @pl.loop(0, nwords // zchunk)
    def _(i):
        pltpu.make_async_copy(zbuf.at[pl.ds(0, zchunk)], acc.at[pl.ds(base + i * zchunk, zchunk)], sem).wait()


def _fire_scatter(nrows, idx_b, val_b, acc, sem):
    @pl.loop(0, nrows)
    def _(r):
        pltpu.async_copy(val_b.at[r], acc.at[idx_b.at[r]], sem, add=True)


def _drain_scatter(nrows, idx_b, val_b, acc, sem):
    @pl.loop(0, nrows)
    def _(r):
        pltpu.make_async_copy(val_b.at[r], acc.at[idx_b.at[r]], sem).wait()


def _copy_out_async(bounce0, bounce1, acc, out_hbm, tile_words, spmem_base, hbm_base, chunk, sem):
    nch = tile_words // chunk

    @pl.loop(0, nch // 2)
    def _(jj):
        for half in range(2):
            buf = bounce0 if half == 0 else bounce1
            i = jj * 2 + half

            @pl.when(i >= 2)
            def _(i=i, buf=buf):
                pltpu.make_async_copy(
                    buf, out_hbm.at[pl.ds(hbm_base + (i - 2) * chunk, chunk)], sem).wait()

            pltpu.sync_copy(acc.at[pl.ds(spmem_base + i * chunk, chunk)], buf)
            pltpu.async_copy(buf, out_hbm.at[pl.ds(hbm_base + i * chunk, chunk)], sem)

    for half in range(2):
        buf = bounce0 if half == 0 else bounce1
        i = nch - 2 + half
        pltpu.make_async_copy(
            buf, out_hbm.at[pl.ds(hbm_base + i * chunk, chunk)], sem).wait()


NBLK_S = 10


@functools.partial(
    pl.kernel,
    out_type=(
        jax.ShapeDtypeStruct((N_GRAPHS,), _f32),
        jax.ShapeDtypeStruct((NP,), _f32),
        jax.ShapeDtypeStruct((2 * SFLAT,), _f32),
    ),
    mesh=_mesh,
    compiler_params=_sc_params,
    scratch_types=[
        pltpu.VMEM((NP,), _f32),
        pltpu.VMEM((NP,), _i32),
        pltpu.VMEM((EBLK,), _i32),
        pltpu.VMEM((EBLK,), _i32),
        pltpu.VMEM((EBLK,), _i32),
        pltpu.VMEM((EBLK,), _i32),
        pltpu.VMEM((ROWS, 128), _i32),
        pltpu.VMEM((ROWS, 128), _f32),
        pltpu.VMEM((4096,), _f32),
        pltpu.VMEM((4096,), _f32),
        pltpu.VMEM_SHARED((SFLAT + DEGW,), _f32),
        pltpu.SemaphoreType.DMA,
        pltpu.SemaphoreType.DMA,
        pltpu.SemaphoreType.DMA,
        pltpu.SemaphoreType.DMA,
    ],
)
def _sc_smat(src_hbm, dst_hbm, batch_hbm, cnt_hbm, dinv_hbm, out_hbm,
             dinv_t, batch_t, src_t0, src_t1, dst_t0, dst_t1, idx_b, val_b,
             bounce0, bounce1, acc,
             sem_z, sem_e, sem_s, sem_o):
    sc = lax.axis_index("c")
    s = lax.axis_index("s")
    ebase = s * 20000
    half_lo = sc * HALF

    _fill_zbuf(bounce0, 4096)
    _fire_zero(bounce0, acc, s * (SFLAT // 16), SFLAT // 16, 4096, sem_z)
    _fire_zero(bounce0, acc, SFLAT + s * (DEGW // 16), DEGW // 16, 656, sem_z)
    pltpu.sync_copy(batch_hbm, batch_t)
    pltpu.async_copy(dst_hbm.at[pl.ds(ebase, EBLK)], dst_t0, sem_e)
    _drain_zero(bounce0, acc, s * (SFLAT // 16), SFLAT // 16, 4096, sem_z)
    _drain_zero(bounce0, acc, SFLAT + s * (DEGW // 16), DEGW // 16, 656, sem_z)
    plsc.subcore_barrier()

    @pl.loop(0, NBLK_S // 2)
    def _(jj):
        for half in range(2):
            ib = jj * 2 + half
            cur_d = dst_t0 if half == 0 else dst_t1
            nxt_d = dst_t1 if half == 0 else dst_t0
            pltpu.make_async_copy(dst_hbm.at[pl.ds(ebase + ib * EBLK, EBLK)], cur_d, sem_e).wait()

            @pl.when(ib + 1 < NBLK_S)
            def _(ib=ib, nxt_d=nxt_d):
                pltpu.async_copy(dst_hbm.at[pl.ds(ebase + (ib + 1) * EBLK, EBLK)], nxt_d, sem_e)

            @pl.loop(0, ROWS)
            def _(r, ib=ib, cur_d=cur_d):
                for k in range(8):
                    off = r * 128 + k * 16
                    lane = ib * EBLK + off + _iota16()
                    m = lane < 20000
                    d16 = cur_d[pl.ds(off, 16)]
                    idx_b[r, pl.ds(k * 16, 16)] = SFLAT + d16
                    val_b[r, pl.ds(k * 16, 16)] = jnp.where(m, 1.0, 0.0).astype(_f32)

            _fire_scatter(ROWS, idx_b, val_b, acc, sem_s)
            _drain_scatter(ROWS, idx_b, val_b, acc, sem_s)

    @pl.loop(0, 5)
    def _(r):
        for k in range(8):
            off = r * 128 + k * 16
            nv = s * 640 + off + _iota16()
            m = nv < N_NODES
            b16 = plsc.load_gather(batch_t, [nv])
            idx_b[r, pl.ds(k * 16, 16)] = SFLAT + NP + b16
            val_b[r, pl.ds(k * 16, 16)] = jnp.where(m, 1.0, 0.0).astype(_f32)

    _fire_scatter(5, idx_b, val_b, acc, sem_s)
    _drain_scatter(5, idx_b, val_b, acc, sem_s)
    plsc.subcore_barrier()

    pltpu.async_copy(src_hbm.at[pl.ds(ebase, EBLK)], src_t0, sem_e)
    pltpu.async_copy(dst_hbm.at[pl.ds(ebase, EBLK)], dst_t0, sem_e)
    pltpu.sync_copy(acc.at[pl.ds(SFLAT, NP)], dinv_t)

    @pl.loop(0, NP // 16)
    def _(i):
        x = dinv_t[pl.ds(i * 16, 16)] + 1.0
        dinv_t[pl.ds(i * 16, 16)] = _rsqrt16(x)

    @pl.when(sc == 0)
    def _():
        pltpu.sync_copy(dinv_t.at[pl.ds(s * 640, 640)], dinv_hbm.at[pl.ds(s * 640, 640)])

        @pl.when(s == 0)
        def _():
            pltpu.sync_copy(acc.at[pl.ds(SFLAT + NP, N_GRAPHS)], bounce1.at[pl.ds(0, N_GRAPHS)])
            pltpu.sync_copy(bounce1.at[pl.ds(0, N_GRAPHS)], cnt_hbm)

    @pl.loop(0, NBLK_S // 2)
    def _(jj):
        for half in range(2):
            ib = jj * 2 + half
            cur_s = src_t0 if half == 0 else src_t1
            cur_d = dst_t0 if half == 0 else dst_t1
            nxt_s = src_t1 if half == 0 else src_t0
            nxt_d = dst_t1 if half == 0 else dst_t0
            pltpu.make_async_copy(src_hbm.at[pl.ds(ebase + ib * EBLK, EBLK)], cur_s, sem_e).wait()
            pltpu.make_async_copy(dst_hbm.at[pl.ds(ebase + ib * EBLK, EBLK)], cur_d, sem_e).wait()

            @pl.when(ib + 1 < NBLK_S)
            def _(ib=ib, nxt_s=nxt_s, nxt_d=nxt_d):
                pltpu.async_copy(src_hbm.at[pl.ds(ebase + (ib + 1) * EBLK, EBLK)], nxt_s, sem_e)
                pltpu.async_copy(dst_hbm.at[pl.ds(ebase + (ib + 1) * EBLK, EBLK)], nxt_d, sem_e)

            @pl.loop(0, ROWS)
            def _(r, ib=ib, cur_s=cur_s, cur_d=cur_d):
                for k in range(8):
                    off = r * 128 + k * 16
                    lane = ib * EBLK + off + _iota16()
                    s16 = cur_s[pl.ds(off, 16)]
                    d16 = cur_d[pl.ds(off, 16)]
                    loc = s16 - half_lo
                    m = (lane < 20000) & (loc >= 0) & (loc < HALF)
                    nrm = plsc.load_gather(dinv_t, [s16]) * plsc.load_gather(dinv_t, [d16])
                    b16 = plsc.load_gather(batch_t, [d16])
                    locc = jnp.where(m, loc, s16 & 4095)
                    idx_b[r, pl.ds(k * 16, 16)] = locc * N_GRAPHS + b16
                    val_b[r, pl.ds(k * 16, 16)] = jnp.where(m, nrm, 0.0)

            _fire_scatter(ROWS, idx_b, val_b, acc, sem_s)
            _drain_scatter(ROWS, idx_b, val_b, acc, sem_s)

    nbase = half_lo + s * 313

    @pl.loop(0, 3)
    def _(r):
        for k in range(8):
            off = r * 128 + k * 16
            lane = off + _iota16()
            nv = nbase + lane
            loc = nv - half_lo
            m = (lane < 313) & (loc < HALF)
            dv = plsc.load_gather(dinv_t, [nv])
            b16 = plsc.load_gather(batch_t, [nv])
            locc = jnp.where(m, loc, nv & 4095)
            idx_b[r, pl.ds(k * 16, 16)] = locc * N_GRAPHS + b16
            val_b[r, pl.ds(k * 16, 16)] = jnp.where(m, dv * dv, 0.0)

    _fire_scatter(3, idx_b, val_b, acc, sem_s)
    _drain_scatter(3, idx_b, val_b, acc, sem_s)
    plsc.subcore_barrier()

    out_w = SFLAT // 16
    _copy_out_async(bounce0, bounce1, acc, out_hbm, out_w, s * out_w, sc * SFLAT + s * out_w, 4096, sem_o)


NBLK_B = 6


@functools.partial(
    pl.kernel,
    out_type=jax.ShapeDtypeStruct((2 * CFLAT,), _f32),
    mesh=_mesh,
    compiler_params=_sc_params,
    scratch_types=[
        pltpu.VMEM((NP,), _f32),
        pltpu.VMEM((NP,), _i32),
        pltpu.VMEM((EBLK,), _i32),
        pltpu.VMEM((EBLK,), _i32),
        pltpu.VMEM((EBLK,), _i32),
        pltpu.VMEM((EBLK,), _i32),
        pltpu.VMEM((ROWS, 128), _i32),
        pltpu.VMEM((ROWS, 128), _f32),
        pltpu.VMEM((4096,), _f32),
        pltpu.VMEM((4096,), _f32),
        pltpu.VMEM_SHARED((CFLAT,), _f32),
        pltpu.SemaphoreType.DMA,
        pltpu.SemaphoreType.DMA,
        pltpu.SemaphoreType.DMA,
        pltpu.SemaphoreType.DMA,
    ],
)
def _sc_cmat(src_hbm, dst_hbm, dinv_hbm, cls_hbm, out_hbm,
             dinv_t, cls_t, src_t0, src_t1, dst_t0, dst_t1, idx_b, val_b,
             bounce0, bounce1, acc,
             sem_z, sem_e, sem_s, sem_o):
    sc = lax.axis_index("c")
    s = lax.axis_index("s")
    w = sc * 16 + s
    ebase = w * 10000
    nbase = w * 313

    _fill_zbuf(bounce0, 4096)
    _fire_zero(bounce0, acc, s * (CFLAT // 16), CFLAT // 16, 4096, sem_z)
    pltpu.sync_copy(dinv_hbm, dinv_t)
    pltpu.sync_copy(cls_hbm, cls_t)
    pltpu.async_copy(src_hbm.at[pl.ds(ebase, EBLK)], src_t0, sem_e)
    pltpu.async_copy(dst_hbm.at[pl.ds(ebase, EBLK)], dst_t0, sem_e)
    _drain_zero(bounce0, acc, s * (CFLAT // 16), CFLAT // 16, 4096, sem_z)
    plsc.subcore_barrier()

    @pl.loop(0, NBLK_B // 2)
    def _(jj):
        for half in range(2):
            ib = jj * 2 + half
            cur_s = src_t0 if half == 0 else src_t1
            cur_d = dst_t0 if half == 0 else dst_t1
            nxt_s = src_t1 if half == 0 else src_t0
            nxt_d = dst_t1 if half == 0 else dst_t0
            pltpu.make_async_copy(src_hbm.at[pl.ds(ebase + ib * EBLK, EBLK)], cur_s, sem_e).wait()
            pltpu.make_async_copy(dst_hbm.at[pl.ds(ebase + ib * EBLK, EBLK)], cur_d, sem_e).wait()

            @pl.when(ib + 1 < NBLK_B)
            def _(ib=ib, nxt_s=nxt_s, nxt_d=nxt_d):
                pltpu.async_copy(src_hbm.at[pl.ds(ebase + (ib + 1) * EBLK, EBLK)], nxt_s, sem_e)
                pltpu.async_copy(dst_hbm.at[pl.ds(ebase + (ib + 1) * EBLK, EBLK)], nxt_d, sem_e)

            @pl.loop(0, ROWS)
            def _(r, ib=ib, cur_s=cur_s, cur_d=cur_d):
                for k in range(8):
                    off = r * 128 + k * 16
                    lane = ib * EBLK + off + _iota16()
                    m = lane < 10000
                    s16 = cur_s[pl.ds(off, 16)]
                    d16 = cur_d[pl.ds(off, 16)]
                    nrm = plsc.load_gather(dinv_t, [s16]) * plsc.load_gather(dinv_t, [d16])
                    c16 = plsc.load_gather(cls_t, [s16])
                    pnd = jnp.where(d16 >= HALF, d16 + 120, d16)
                    idx_b[r, pl.ds(k * 16, 16)] = pnd * 128 + c16
                    val_b[r, pl.ds(k * 16, 16)] = jnp.where(m, nrm, 0.0)

            _fire_scatter(ROWS, idx_b, val_b, acc, sem_s)
            _drain_scatter(ROWS, idx_b, val_b, acc, sem_s)

    @pl.loop(0, 3)
    def _(r):
        for k in range(8):
            off = r * 128 + k * 16
            lane = off + _iota16()
            nv = nbase + lane
            m = (lane < 313) & (nv < N_NODES)
            dv = plsc.load_gather(dinv_t, [nv])
            c16 = plsc.load_gather(cls_t, [nv])
            pnn = jnp.where(nv >= HALF, nv + 120, nv)
            idx_b[r, pl.ds(k * 16, 16)] = pnn * 128 + c16
            val_b[r, pl.ds(k * 16, 16)] = jnp.where(m, dv * dv, 0.0)

    _fire_scatter(3, idx_b, val_b, acc, sem_s)
    _drain_scatter(3, idx_b, val_b, acc, sem_s)
    plsc.subcore_barrier()

    out_w = CFLAT // 16
    _copy_out_async(bounce0, bounce1, acc, out_hbm, out_w, s * out_w, sc * CFLAT + s * out_w, 4096, sem_o)


KBLK = 256
_bf16 = jnp.bfloat16


def _tc_body(c0_ref, c1_ref, emb_ref, w1_ref, b1_ref, w2_ref, st_ref, cnt_ref,
             b2_ref, out_ref, t1_ref, w2b_ref):
    @pl.when((pl.program_id(0) == 0) & (pl.program_id(1) == 0))
    def _():
        t1_ref[...] = jnp.dot(emb_ref[...], w1_ref[...],
                              preferred_element_type=_f32).astype(_bf16)
        w2b_ref[...] = w2_ref[...].astype(_bf16)
        out_ref[...] = cnt_ref[...] * b2_ref[...]

    a = (c0_ref[...] + c1_ref[...]).astype(_bf16)
    h = jnp.dot(a, t1_ref[...], preferred_element_type=_f32) + b1_ref[...]
    h = jnp.maximum(h, 0.0).astype(_bf16)
    hw = jnp.dot(h, w2b_ref[...], preferred_element_type=_f32).astype(_bf16)
    st = st_ref[...].astype(_bf16)
    out_ref[...] += lax.dot_general(
        st, hw, (((0,), (0,)), ((), ())), preferred_element_type=_f32)


def _tc_dense(c2, embed_p, w1, b1r, w2, st2, cntc, b2r):
    nk = HP // KBLK
    return pl.pallas_call(
        _tc_body,
        grid=(2, nk),
        in_specs=[
            pl.BlockSpec((KBLK, 128), lambda s, k: (s * (HP // KBLK) + k, 0)),
            pl.BlockSpec((KBLK, 128), lambda s, k: (NP // KBLK + s * (HP // KBLK) + k, 0)),
            pl.BlockSpec((128, 128), lambda s, k: (0, 0)),
            pl.BlockSpec((128, 128), lambda s, k: (0, 0)),
            pl.BlockSpec((1, 128), lambda s, k: (0, 0)),
            pl.BlockSpec((128, 128), lambda s, k: (0, 0)),
            pl.BlockSpec((KBLK, N_GRAPHS), lambda s, k: (s * (HP // KBLK) + k, 0)),
            pl.BlockSpec((N_GRAPHS, 1), lambda s, k: (0, 0)),
            pl.BlockSpec((1, 128), lambda s, k: (0, 0)),
        ],
        out_specs=pl.BlockSpec((N_GRAPHS, 128), lambda s, k: (0, 0)),
        out_shape=jax.ShapeDtypeStruct((N_GRAPHS, 128), _f32),
        scratch_shapes=[pltpu.VMEM((128, 128), _bf16), pltpu.VMEM((128, 128), _bf16)],
    )(c2, c2, embed_p, w1, b1r, w2, st2, cntc, b2r)


def kernel(x, edge_index, batch, embed_atom, W1, b1, W2, b2):
    cls_p = jnp.pad(x[:, 0].astype(_i32), (0, NP - N_NODES))
    batch_p = jnp.pad(batch.astype(_i32), (0, NP - N_NODES))
    src_p = jnp.pad(edge_index[0].astype(_i32), (0, EP - N_EDGES))
    dst_p = jnp.pad(edge_index[1].astype(_i32), (0, EP - N_EDGES))
    embed_p = jnp.pad(embed_atom.astype(_f32), ((0, 128 - embed_atom.shape[0]), (0, 0)))
    b1r = b1.reshape(1, HID).astype(_f32)
    b2r = b2.reshape(1, HID).astype(_f32)

    cnt, dinv, smat = _sc_smat(src_p, dst_p, batch_p)
    cmat = _sc_cmat(src_p, dst_p, dinv, cls_p)

    c2 = cmat.reshape(2 * NP, 128)
    st2 = smat.reshape(2 * HP, N_GRAPHS)
    return _tc_dense(c2, embed_p, W1.astype(_f32), b1r,
                     W2.astype(_f32), st2, cnt.reshape(N_GRAPHS, 1), b2r)


# speedup vs baseline: 61.2457x; 1.0555x over previous
import dataclasses
import functools

import jax
import jax.numpy as jnp
from jax import lax
from jax.experimental import pallas as pl
from jax.experimental.pallas import tpu as pltpu
from jax.experimental.pallas import tpu_sc as plsc

N_NODES = 10000
N_EDGES = 320000
HID = 128
N_GRAPHS = 256
NP = 10240
HALF = 5000
HP = 5120
EP = N_EDGES + 4096
DEGW = NP + N_GRAPHS
CFLAT = NP * HID
SFLAT = N_GRAPHS * HP

_mesh = plsc.VectorSubcoreMesh(core_axis_name="c", subcore_axis_name="s")
_sc_params = pltpu.CompilerParams()
if "needs_layout_passes" in pltpu.CompilerParams.__dataclass_fields__:
    _sc_params = dataclasses.replace(_sc_params, needs_layout_passes=False)
_f32 = jnp.float32
_i32 = jnp.int32

EBLK = 2048
ROWS = 16


def _iota16():
    return lax.iota(_i32, 16)


def _rsqrt16(x):
    i = plsc.bitcast(x, _i32)
    i = 0x5F3759DF - lax.shift_right_logical(i, 1)
    y = plsc.bitcast(i, _f32)
    for _ in range(3):
        y = y * (1.5 - 0.5 * x * y * y)
    return y


def _fill_zbuf(zbuf, nwords):
    @pl.loop(0, nwords // 16)
    def _(i):
        zbuf[pl.ds(i * 16, 16)] = jnp.zeros((16,), _f32)


def _fire_zero(zbuf, acc, base, nwords, zchunk, sem):
    @pl.loop(0, nwords // zchunk)
    def _(i):
        pltpu.async_copy(zbuf.at[pl.ds(0, zchunk)], acc.at[pl.ds(base + i * zchunk, zchunk)], sem)


def _drain_zero(zbuf, acc, base, nwords, zchunk, sem):
    @pl.loop(0, nwords // zchunk)
    def _(i):
        pltpu.make_async_copy(zbuf.at[pl.ds(0, zchunk)], acc.at[pl.ds(base + i * zchunk, zchunk)], sem).wait()


def _fire_scatter(nrows, idx_b, val_b, acc, sem):
    @pl.loop(0, nrows)
    def _(r):
        pltpu.async_copy(val_b.at[r], acc.at[idx_b.at[r]], sem, add=True)


def _drain_scatter(nrows, idx_b, val_b, acc, sem):
    @pl.loop(0, nrows)
    def _(r):
        pltpu.make_async_copy(val_b.at[r], acc.at[idx_b.at[r]], sem).wait()


def _copy_out_async(bounce0, bounce1, acc, out_hbm, tile_words, spmem_base, hbm_base, chunk, sem):
    nch = tile_words // chunk

    @pl.loop(0, nch // 2)
    def _(jj):
        for half in range(2):
            buf = bounce0 if half == 0 else bounce1
            i = jj * 2 + half

            @pl.when(i >= 2)
            def _(i=i, buf=buf):
                pltpu.make_async_copy(
                    buf, out_hbm.at[pl.ds(hbm_base + (i - 2) * chunk, chunk)], sem).wait()

            pltpu.sync_copy(acc.at[pl.ds(spmem_base + i * chunk, chunk)], buf)
            pltpu.async_copy(buf, out_hbm.at[pl.ds(hbm_base + i * chunk, chunk)], sem)

    for half in range(2):
        buf = bounce0 if half == 0 else bounce1
        i = nch - 2 + half
        pltpu.make_async_copy(
            buf, out_hbm.at[pl.ds(hbm_base + i * chunk, chunk)], sem).wait()


NBLK_S = 10


@functools.partial(
    pl.kernel,
    out_type=(
        jax.ShapeDtypeStruct((N_GRAPHS,), _f32),
        jax.ShapeDtypeStruct((NP,), _f32),
        jax.ShapeDtypeStruct((2 * SFLAT,), _f32),
    ),
    mesh=_mesh,
    compiler_params=_sc_params,
    scratch_types=[
        pltpu.VMEM((NP,), _f32),
        pltpu.VMEM((NP,), _i32),
        pltpu.VMEM((EBLK,), _i32),
        pltpu.VMEM((EBLK,), _i32),
        pltpu.VMEM((EBLK,), _i32),
        pltpu.VMEM((EBLK,), _i32),
        pltpu.VMEM((ROWS, 128), _i32),
        pltpu.VMEM((ROWS, 128), _f32),
        pltpu.VMEM((4096,), _f32),
        pltpu.VMEM((4096,), _f32),
        pltpu.VMEM_SHARED((SFLAT + DEGW,), _f32),
        pltpu.SemaphoreType.DMA,
        pltpu.SemaphoreType.DMA,
        pltpu.SemaphoreType.DMA,
        pltpu.SemaphoreType.DMA,
    ],
)
def _sc_smat(ei_hbm, batch_hbm, cnt_hbm, dinv_hbm, out_hbm,
             dinv_t, batch_t, src_t0, src_t1, dst_t0, dst_t1, idx_b, val_b,
             bounce0, bounce1, acc,
             sem_z, sem_e, sem_s, sem_o):
    sc = lax.axis_index("c")
    s = lax.axis_index("s")
    ebase = s * 20000
    half_lo = sc * HALF

    _fill_zbuf(bounce0, 4096)
    _fire_zero(bounce0, acc, s * (SFLAT // 16), SFLAT // 16, 4096, sem_z)
    _fire_zero(bounce0, acc, SFLAT + s * (DEGW // 16), DEGW // 16, 656, sem_z)
    pltpu.sync_copy(batch_hbm, batch_t)
    pltpu.async_copy(ei_hbm.at[pl.ds(EP + ebase, EBLK)], dst_t0, sem_e)
    _drain_zero(bounce0, acc, s * (SFLAT // 16), SFLAT // 16, 4096, sem_z)
    _drain_zero(bounce0, acc, SFLAT + s * (DEGW // 16), DEGW // 16, 656, sem_z)
    plsc.subcore_barrier()

    @pl.loop(0, NBLK_S // 2)
    def _(jj):
        for half in range(2):
            ib = jj * 2 + half
            cur_d = dst_t0 if half == 0 else dst_t1
            nxt_d = dst_t1 if half == 0 else dst_t0
            pltpu.make_async_copy(ei_hbm.at[pl.ds(EP + ebase + ib * EBLK, EBLK)], cur_d, sem_e).wait()

            @pl.when(ib + 1 < NBLK_S)
            def _(ib=ib, nxt_d=nxt_d):
                pltpu.async_copy(ei_hbm.at[pl.ds(EP + ebase + (ib + 1) * EBLK, EBLK)], nxt_d, sem_e)

            @pl.loop(0, ROWS)
            def _(r, ib=ib, cur_d=cur_d):
                for k in range(8):
                    off = r * 128 + k * 16
                    lane = ib * EBLK + off + _iota16()
                    m = lane < 20000
                    d16 = cur_d[pl.ds(off, 16)]
                    idx_b[r, pl.ds(k * 16, 16)] = SFLAT + d16
                    val_b[r, pl.ds(k * 16, 16)] = jnp.where(m, 1.0, 0.0).astype(_f32)

            _fire_scatter(ROWS, idx_b, val_b, acc, sem_s)
            _drain_scatter(ROWS, idx_b, val_b, acc, sem_s)

    @pl.loop(0, 5)
    def _(r):
        for k in range(8):
            off = r * 128 + k * 16
            nv = s * 640 + off + _iota16()
            m = nv < N_NODES
            b16 = plsc.load_gather(batch_t, [nv])
            idx_b[r, pl.ds(k * 16, 16)] = SFLAT + NP + b16
            val_b[r, pl.ds(k * 16, 16)] = jnp.where(m, 1.0, 0.0).astype(_f32)

    _fire_scatter(5, idx_b, val_b, acc, sem_s)
    _drain_scatter(5, idx_b, val_b, acc, sem_s)
    plsc.subcore_barrier()

    pltpu.async_copy(ei_hbm.at[pl.ds(ebase, EBLK)], src_t0, sem_e)
    pltpu.async_copy(ei_hbm.at[pl.ds(EP + ebase, EBLK)], dst_t0, sem_e)
    pltpu.sync_copy(acc.at[pl.ds(SFLAT, NP)], dinv_t)

    @pl.loop(0, NP // 16)
    def _(i):
        x = dinv_t[pl.ds(i * 16, 16)] + 1.0
        dinv_t[pl.ds(i * 16, 16)] = _rsqrt16(x)

    @pl.when(sc == 0)
    def _():
        pltpu.sync_copy(dinv_t.at[pl.ds(s * 640, 640)], dinv_hbm.at[pl.ds(s * 640, 640)])

        @pl.when(s == 0)
        def _():
            pltpu.sync_copy(acc.at[pl.ds(SFLAT + NP, N_GRAPHS)], bounce1.at[pl.ds(0, N_GRAPHS)])
            pltpu.sync_copy(bounce1.at[pl.ds(0, N_GRAPHS)], cnt_hbm)

    @pl.loop(0, NBLK_S // 2)
    def _(jj):
        for half in range(2):
            ib = jj * 2 + half
            cur_s = src_t0 if half == 0 else src_t1
            cur_d = dst_t0 if half == 0 else dst_t1
            nxt_s = src_t1 if half == 0 else src_t0
            nxt_d = dst_t1 if half == 0 else dst_t0
            pltpu.make_async_copy(ei_hbm.at[pl.ds(ebase + ib * EBLK, EBLK)], cur_s, sem_e).wait()
            pltpu.make_async_copy(ei_hbm.at[pl.ds(EP + ebase + ib * EBLK, EBLK)], cur_d, sem_e).wait()

            @pl.when(ib + 1 < NBLK_S)
            def _(ib=ib, nxt_s=nxt_s, nxt_d=nxt_d):
                pltpu.async_copy(ei_hbm.at[pl.ds(ebase + (ib + 1) * EBLK, EBLK)], nxt_s, sem_e)
                pltpu.async_copy(ei_hbm.at[pl.ds(EP + ebase + (ib + 1) * EBLK, EBLK)], nxt_d, sem_e)

            @pl.loop(0, ROWS)
            def _(r, ib=ib, cur_s=cur_s, cur_d=cur_d):
                for k in range(8):
                    off = r * 128 + k * 16
                    lane = ib * EBLK + off + _iota16()
                    s16 = cur_s[pl.ds(off, 16)]
                    d16 = cur_d[pl.ds(off, 16)]
                    loc = s16 - half_lo
                    m = (lane < 20000) & (loc >= 0) & (loc < HALF)
                    nrm = plsc.load_gather(dinv_t, [s16]) * plsc.load_gather(dinv_t, [d16])
                    b16 = plsc.load_gather(batch_t, [d16])
                    locc = jnp.where(m, loc, s16 & 4095)
                    idx_b[r, pl.ds(k * 16, 16)] = locc * N_GRAPHS + b16
                    val_b[r, pl.ds(k * 16, 16)] = jnp.where(m, nrm, 0.0)

            _fire_scatter(ROWS, idx_b, val_b, acc, sem_s)
            _drain_scatter(ROWS, idx_b, val_b, acc, sem_s)

    nbase = half_lo + s * 313

    @pl.loop(0, 3)
    def _(r):
        for k in range(8):
            off = r * 128 + k * 16
            lane = off + _iota16()
            nv = nbase + lane
            loc = nv - half_lo
            m = (lane < 313) & (loc < HALF)
            dv = plsc.load_gather(dinv_t, [nv])
            b16 = plsc.load_gather(batch_t, [nv])
            locc = jnp.where(m, loc, nv & 4095)
            idx_b[r, pl.ds(k * 16, 16)] = locc * N_GRAPHS + b16
            val_b[r, pl.ds(k * 16, 16)] = jnp.where(m, dv * dv, 0.0)

    _fire_scatter(3, idx_b, val_b, acc, sem_s)
    _drain_scatter(3, idx_b, val_b, acc, sem_s)
    plsc.subcore_barrier()

    out_w = SFLAT // 16
    _copy_out_async(bounce0, bounce1, acc, out_hbm, out_w, s * out_w, sc * SFLAT + s * out_w, 4096, sem_o)


NBLK_B = 6


@functools.partial(
    pl.kernel,
    out_type=jax.ShapeDtypeStruct((2 * CFLAT,), _f32),
    mesh=_mesh,
    compiler_params=_sc_params,
    scratch_types=[
        pltpu.VMEM((NP,), _f32),
        pltpu.VMEM((NP,), _i32),
        pltpu.VMEM((EBLK,), _i32),
        pltpu.VMEM((EBLK,), _i32),
        pltpu.VMEM((EBLK,), _i32),
        pltpu.VMEM((EBLK,), _i32),
        pltpu.VMEM((ROWS, 128), _i32),
        pltpu.VMEM((ROWS, 128), _f32),
        pltpu.VMEM((4096,), _f32),
        pltpu.VMEM((4096,), _f32),
        pltpu.VMEM_SHARED((CFLAT,), _f32),
        pltpu.SemaphoreType.DMA,
        pltpu.SemaphoreType.DMA,
        pltpu.SemaphoreType.DMA,
        pltpu.SemaphoreType.DMA,
    ],
)
def _sc_cmat(ei_hbm, dinv_hbm, cls_hbm, out_hbm,
             dinv_t, cls_t, src_t0, src_t1, dst_t0, dst_t1, idx_b, val_b,
             bounce0, bounce1, acc,
             sem_z, sem_e, sem_s, sem_o):
    sc = lax.axis_index("c")
    s = lax.axis_index("s")
    w = sc * 16 + s
    ebase = w * 10000
    nbase = w * 313

    _fill_zbuf(bounce0, 4096)
    _fire_zero(bounce0, acc, s * (CFLAT // 16), CFLAT // 16, 4096, sem_z)
    pltpu.sync_copy(dinv_hbm, dinv_t)
    pltpu.sync_copy(cls_hbm, cls_t)
    pltpu.async_copy(ei_hbm.at[pl.ds(ebase, EBLK)], src_t0, sem_e)
    pltpu.async_copy(ei_hbm.at[pl.ds(EP + ebase, EBLK)], dst_t0, sem_e)
    _drain_zero(bounce0, acc, s * (CFLAT // 16), CFLAT // 16, 4096, sem_z)
    plsc.subcore_barrier()

    @pl.loop(0, NBLK_B // 2)
    def _(jj):
        for half in range(2):
            ib = jj * 2 + half
            cur_s = src_t0 if half == 0 else src_t1
            cur_d = dst_t0 if half == 0 else dst_t1
            nxt_s = src_t1 if half == 0 else src_t0
            nxt_d = dst_t1 if half == 0 else dst_t0
            pltpu.make_async_copy(ei_hbm.at[pl.ds(ebase + ib * EBLK, EBLK)], cur_s, sem_e).wait()
            pltpu.make_async_copy(ei_hbm.at[pl.ds(EP + ebase + ib * EBLK, EBLK)], cur_d, sem_e).wait()

            @pl.when(ib + 1 < NBLK_B)
            def _(ib=ib, nxt_s=nxt_s, nxt_d=nxt_d):
                pltpu.async_copy(ei_hbm.at[pl.ds(ebase + (ib + 1) * EBLK, EBLK)], nxt_s, sem_e)
                pltpu.async_copy(ei_hbm.at[pl.ds(EP + ebase + (ib + 1) * EBLK, EBLK)], nxt_d, sem_e)

            @pl.loop(0, ROWS)
            def _(r, ib=ib, cur_s=cur_s, cur_d=cur_d):
                for k in range(8):
                    off = r * 128 + k * 16
                    lane = ib * EBLK + off + _iota16()
                    m = lane < 10000
                    s16 = cur_s[pl.ds(off, 16)]
                    d16 = cur_d[pl.ds(off, 16)]
                    nrm = plsc.load_gather(dinv_t, [s16]) * plsc.load_gather(dinv_t, [d16])
                    c16 = plsc.load_gather(cls_t, [s16])
                    pnd = jnp.where(d16 >= HALF, d16 + 120, d16)
                    idx_b[r, pl.ds(k * 16, 16)] = pnd * 128 + c16
                    val_b[r, pl.ds(k * 16, 16)] = jnp.where(m, nrm, 0.0)

            _fire_scatter(ROWS, idx_b, val_b, acc, sem_s)
            _drain_scatter(ROWS, idx_b, val_b, acc, sem_s)

    @pl.loop(0, 3)
    def _(r):
        for k in range(8):
            off = r * 128 + k * 16
            lane = off + _iota16()
            nv = nbase + lane
            m = (lane < 313) & (nv < N_NODES)
            dv = plsc.load_gather(dinv_t, [nv])
            c16 = plsc.load_gather(cls_t, [nv])
            pnn = jnp.where(nv >= HALF, nv + 120, nv)
            idx_b[r, pl.ds(k * 16, 16)] = pnn * 128 + c16
            val_b[r, pl.ds(k * 16, 16)] = jnp.where(m, dv * dv, 0.0)

    _fire_scatter(3, idx_b, val_b, acc, sem_s)
    _drain_scatter(3, idx_b, val_b, acc, sem_s)
    plsc.subcore_barrier()

    out_w = CFLAT // 16
    _copy_out_async(bounce0, bounce1, acc, out_hbm, out_w, s * out_w, sc * CFLAT + s * out_w, 4096, sem_o)


KBLK = 256
_bf16 = jnp.bfloat16


def _tc_body(c0_ref, c1_ref, emb_ref, w1_ref, b1_ref, w2_ref, st_ref, cnt_ref,
             b2_ref, out_ref, t1_ref, w2b_ref):
    @pl.when((pl.program_id(0) == 0) & (pl.program_id(1) == 0))
    def _():
        t1_ref[...] = jnp.dot(emb_ref[...], w1_ref[...],
                              preferred_element_type=_f32).astype(_bf16)
        w2b_ref[...] = w2_ref[...].astype(_bf16)
        out_ref[...] = cnt_ref[...] * b2_ref[...]

    a = (c0_ref[...] + c1_ref[...]).astype(_bf16)
    h = jnp.dot(a, t1_ref[...], preferred_element_type=_f32) + b1_ref[...]
    h = jnp.maximum(h, 0.0).astype(_bf16)
    hw = jnp.dot(h, w2b_ref[...], preferred_element_type=_f32).astype(_bf16)
    st = st_ref[...].astype(_bf16)
    out_ref[...] += lax.dot_general(
        st, hw, (((0,), (0,)), ((), ())), preferred_element_type=_f32)


def _tc_dense(c2, embed_p, w1, b1r, w2, st2, cntc, b2r):
    nk = HP // KBLK
    return pl.pallas_call(
        _tc_body,
        grid=(2, nk),
        in_specs=[
            pl.BlockSpec((KBLK, 128), lambda s, k: (s * (HP // KBLK) + k, 0)),
            pl.BlockSpec((KBLK, 128), lambda s, k: (NP // KBLK + s * (HP // KBLK) + k, 0)),
            pl.BlockSpec((128, 128), lambda s, k: (0, 0)),
            pl.BlockSpec((128, 128), lambda s, k: (0, 0)),
            pl.BlockSpec((1, 128), lambda s, k: (0, 0)),
            pl.BlockSpec((128, 128), lambda s, k: (0, 0)),
            pl.BlockSpec((KBLK, N_GRAPHS), lambda s, k: (s * (HP // KBLK) + k, 0)),
            pl.BlockSpec((N_GRAPHS, 1), lambda s, k: (0, 0)),
            pl.BlockSpec((1, 128), lambda s, k: (0, 0)),
        ],
        out_specs=pl.BlockSpec((N_GRAPHS, 128), lambda s, k: (0, 0)),
        out_shape=jax.ShapeDtypeStruct((N_GRAPHS, 128), _f32),
        scratch_shapes=[pltpu.VMEM((128, 128), _bf16), pltpu.VMEM((128, 128), _bf16)],
    )(c2, c2, embed_p, w1, b1r, w2, st2, cntc, b2r)


def kernel(x, edge_index, batch, embed_atom, W1, b1, W2, b2):
    cls_p = jnp.pad(x[:, 0].astype(_i32), (0, NP - N_NODES))
    batch_p = jnp.pad(batch.astype(_i32), (0, NP - N_NODES))
    ei_p = jnp.pad(edge_index.astype(_i32), ((0, 0), (0, EP - N_EDGES))).reshape(2 * EP)
    embed_p = jnp.pad(embed_atom.astype(_f32), ((0, 128 - embed_atom.shape[0]), (0, 0)))
    b1r = b1.reshape(1, HID).astype(_f32)
    b2r = b2.reshape(1, HID).astype(_f32)

    cnt, dinv, smat = _sc_smat(ei_p, batch_p)
    cmat = _sc_cmat(ei_p, dinv, cls_p)

    c2 = cmat.reshape(2 * NP, 128)
    st2 = smat.reshape(2 * HP, N_GRAPHS)
    return _tc_dense(c2, embed_p, W1.astype(_f32), b1r,
                     W2.astype(_f32), st2, cnt.reshape(N_GRAPHS, 1), b2r)


# speedup vs baseline: 64.2201x; 1.0486x over previous
import dataclasses
import functools

import jax
import jax.numpy as jnp
from jax import lax
from jax.experimental import pallas as pl
from jax.experimental.pallas import tpu as pltpu
from jax.experimental.pallas import tpu_sc as plsc

N_NODES = 10000
N_EDGES = 320000
HID = 128
N_GRAPHS = 256
NP = 10240
HALF = 5000
HP = 5120
EP = N_EDGES + 4096
DEGW = NP + N_GRAPHS
CFLAT = NP * HID
SFLAT = N_GRAPHS * HP

_mesh = plsc.VectorSubcoreMesh(core_axis_name="c", subcore_axis_name="s")
_sc_params = pltpu.CompilerParams()
if "needs_layout_passes" in pltpu.CompilerParams.__dataclass_fields__:
    _sc_params = dataclasses.replace(_sc_params, needs_layout_passes=False)
_f32 = jnp.float32
_i32 = jnp.int32

EBLK = 2048
ROWS = 16


def _iota16():
    return lax.iota(_i32, 16)


def _rsqrt16(x):
    i = plsc.bitcast(x, _i32)
    i = 0x5F3759DF - lax.shift_right_logical(i, 1)
    y = plsc.bitcast(i, _f32)
    for _ in range(3):
        y = y * (1.5 - 0.5 * x * y * y)
    return y


def _fill_zbuf(zbuf, nwords):
    @pl.loop(0, nwords // 16)
    def _(i):
        zbuf[pl.ds(i * 16, 16)] = jnp.zeros((16,), _f32)


def _fire_zero(zbuf, acc, base, nwords, zchunk, sem):
    @pl.loop(0, nwords // zchunk)
    def _(i):
        pltpu.async_copy(zbuf.at[pl.ds(0, zchunk)], acc.at[pl.ds(base + i * zchunk, zchunk)], sem)


def _drain_zero(zbuf, acc, base, nwords, zchunk, sem):
    @pl.loop(0, nwords // zchunk)
    def _(i):
        pltpu.make_async_copy(zbuf.at[pl.ds(0, zchunk)], acc.at[pl.ds(base + i * zchunk, zchunk)], sem).wait()


def _fire_scatter(nrows, idx_b, val_b, acc, sem):
    @pl.loop(0, nrows)
    def _(r):
        pltpu.async_copy(val_b.at[r], acc.at[idx_b.at[r]], sem, add=True)


def _drain_scatter(nrows, idx_b, val_b, acc, sem):
    @pl.loop(0, nrows)
    def _(r):
        pltpu.make_async_copy(val_b.at[r], acc.at[idx_b.at[r]], sem).wait()


def _copy_out_async(bounce0, bounce1, acc, out_hbm, tile_words, spmem_base, hbm_base, chunk, sem):
    nch = tile_words // chunk

    @pl.loop(0, nch // 2)
    def _(jj):
        for half in range(2):
            buf = bounce0 if half == 0 else bounce1
            i = jj * 2 + half

            @pl.when(i >= 2)
            def _(i=i, buf=buf):
                pltpu.make_async_copy(
                    buf, out_hbm.at[pl.ds(hbm_base + (i - 2) * chunk, chunk)], sem).wait()

            pltpu.sync_copy(acc.at[pl.ds(spmem_base + i * chunk, chunk)], buf)
            pltpu.async_copy(buf, out_hbm.at[pl.ds(hbm_base + i * chunk, chunk)], sem)

    for half in range(2):
        buf = bounce0 if half == 0 else bounce1
        i = nch - 2 + half
        pltpu.make_async_copy(
            buf, out_hbm.at[pl.ds(hbm_base + i * chunk, chunk)], sem).wait()


NBLK_S = 10


@functools.partial(
    pl.kernel,
    out_type=(
        jax.ShapeDtypeStruct((N_GRAPHS,), _f32),
        jax.ShapeDtypeStruct((NP,), _f32),
        jax.ShapeDtypeStruct((2 * SFLAT,), _f32),
    ),
    mesh=_mesh,
    compiler_params=_sc_params,
    scratch_types=[
        pltpu.VMEM((NP,), _f32),
        pltpu.VMEM((NP,), _i32),
        pltpu.VMEM((EBLK,), _i32),
        pltpu.VMEM((EBLK,), _i32),
        pltpu.VMEM((EBLK,), _i32),
        pltpu.VMEM((EBLK,), _i32),
        pltpu.VMEM((ROWS, 128), _i32),
        pltpu.VMEM((ROWS, 128), _f32),
        pltpu.VMEM((ROWS, 128), _i32),
        pltpu.VMEM((ROWS, 128), _f32),
        pltpu.VMEM((2048,), _f32),
        pltpu.VMEM((2048,), _f32),
        pltpu.VMEM_SHARED((SFLAT + DEGW,), _f32),
        pltpu.SemaphoreType.DMA,
        pltpu.SemaphoreType.DMA,
        pltpu.SemaphoreType.DMA,
        pltpu.SemaphoreType.DMA,
        pltpu.SemaphoreType.DMA,
    ],
)
def _sc_smat(ei_hbm, batch_hbm, cnt_hbm, dinv_hbm, out_hbm,
             dinv_t, batch_t, src_t0, src_t1, dst_t0, dst_t1, idx_b, val_b,
             idx_b2, val_b2, bounce0, bounce1, acc,
             sem_z, sem_e, sem_s, sem_s2, sem_o):
    sc = lax.axis_index("c")
    s = lax.axis_index("s")
    ebase = s * 20000
    half_lo = sc * HALF

    _fill_zbuf(bounce0, 2048)
    _fire_zero(bounce0, acc, s * (SFLAT // 16), SFLAT // 16, 2048, sem_z)
    _fire_zero(bounce0, acc, SFLAT + s * (DEGW // 16), DEGW // 16, 656, sem_z)
    pltpu.sync_copy(batch_hbm, batch_t)
    pltpu.async_copy(ei_hbm.at[pl.ds(EP + ebase, EBLK)], dst_t0, sem_e)
    _drain_zero(bounce0, acc, s * (SFLAT // 16), SFLAT // 16, 2048, sem_z)
    _drain_zero(bounce0, acc, SFLAT + s * (DEGW // 16), DEGW // 16, 656, sem_z)
    plsc.subcore_barrier()

    @pl.loop(0, NBLK_S // 2)
    def _(jj):
        for half in range(2):
            ib = jj * 2 + half
            cur_d = dst_t0 if half == 0 else dst_t1
            nxt_d = dst_t1 if half == 0 else dst_t0
            cur_i = idx_b if half == 0 else idx_b2
            cur_v = val_b if half == 0 else val_b2
            cur_sem = sem_s if half == 0 else sem_s2
            pltpu.make_async_copy(ei_hbm.at[pl.ds(EP + ebase + ib * EBLK, EBLK)], cur_d, sem_e).wait()

            @pl.when(ib + 1 < NBLK_S)
            def _(ib=ib, nxt_d=nxt_d):
                pltpu.async_copy(ei_hbm.at[pl.ds(EP + ebase + (ib + 1) * EBLK, EBLK)], nxt_d, sem_e)

            @pl.when(ib >= 2)
            def _(cur_i=cur_i, cur_v=cur_v, cur_sem=cur_sem):
                _drain_scatter(ROWS, cur_i, cur_v, acc, cur_sem)

            @pl.loop(0, ROWS)
            def _(r, ib=ib, cur_d=cur_d, cur_i=cur_i, cur_v=cur_v):
                for k in range(8):
                    off = r * 128 + k * 16
                    lane = ib * EBLK + off + _iota16()
                    m = lane < 20000
                    d16 = cur_d[pl.ds(off, 16)]
                    cur_i[r, pl.ds(k * 16, 16)] = SFLAT + d16
                    cur_v[r, pl.ds(k * 16, 16)] = jnp.where(m, 1.0, 0.0).astype(_f32)

            _fire_scatter(ROWS, cur_i, cur_v, acc, cur_sem)

    _drain_scatter(ROWS, idx_b, val_b, acc, sem_s)
    _drain_scatter(ROWS, idx_b2, val_b2, acc, sem_s2)

    @pl.loop(0, 5)
    def _(r):
        for k in range(8):
            off = r * 128 + k * 16
            nv = s * 640 + off + _iota16()
            m = nv < N_NODES
            b16 = plsc.load_gather(batch_t, [nv])
            idx_b[r, pl.ds(k * 16, 16)] = SFLAT + NP + b16
            val_b[r, pl.ds(k * 16, 16)] = jnp.where(m, 1.0, 0.0).astype(_f32)

    _fire_scatter(5, idx_b, val_b, acc, sem_s)
    _drain_scatter(5, idx_b, val_b, acc, sem_s)
    plsc.subcore_barrier()

    pltpu.async_copy(ei_hbm.at[pl.ds(ebase, EBLK)], src_t0, sem_e)
    pltpu.async_copy(ei_hbm.at[pl.ds(EP + ebase, EBLK)], dst_t0, sem_e)
    pltpu.sync_copy(acc.at[pl.ds(SFLAT, NP)], dinv_t)

    @pl.loop(0, NP // 16)
    def _(i):
        x = dinv_t[pl.ds(i * 16, 16)] + 1.0
        dinv_t[pl.ds(i * 16, 16)] = _rsqrt16(x)

    @pl.when(sc == 0)
    def _():
        pltpu.sync_copy(dinv_t.at[pl.ds(s * 640, 640)], dinv_hbm.at[pl.ds(s * 640, 640)])

        @pl.when(s == 0)
        def _():
            pltpu.sync_copy(acc.at[pl.ds(SFLAT + NP, N_GRAPHS)], bounce1.at[pl.ds(0, N_GRAPHS)])
            pltpu.sync_copy(bounce1.at[pl.ds(0, N_GRAPHS)], cnt_hbm)

    @pl.loop(0, NBLK_S // 2)
    def _(jj):
        for half in range(2):
            ib = jj * 2 + half
            cur_s = src_t0 if half == 0 else src_t1
            cur_d = dst_t0 if half == 0 else dst_t1
            nxt_s = src_t1 if half == 0 else src_t0
            nxt_d = dst_t1 if half == 0 else dst_t0
            cur_i = idx_b if half == 0 else idx_b2
            cur_v = val_b if half == 0 else val_b2
            cur_sem = sem_s if half == 0 else sem_s2
            pltpu.make_async_copy(ei_hbm.at[pl.ds(ebase + ib * EBLK, EBLK)], cur_s, sem_e).wait()
            pltpu.make_async_copy(ei_hbm.at[pl.ds(EP + ebase + ib * EBLK, EBLK)], cur_d, sem_e).wait()

            @pl.when(ib + 1 < NBLK_S)
            def _(ib=ib, nxt_s=nxt_s, nxt_d=nxt_d):
                pltpu.async_copy(ei_hbm.at[pl.ds(ebase + (ib + 1) * EBLK, EBLK)], nxt_s, sem_e)
                pltpu.async_copy(ei_hbm.at[pl.ds(EP + ebase + (ib + 1) * EBLK, EBLK)], nxt_d, sem_e)

            @pl.when(ib >= 2)
            def _(cur_i=cur_i, cur_v=cur_v, cur_sem=cur_sem):
                _drain_scatter(ROWS, cur_i, cur_v, acc, cur_sem)

            @pl.loop(0, ROWS)
            def _(r, ib=ib, cur_s=cur_s, cur_d=cur_d, cur_i=cur_i, cur_v=cur_v):
                for k in range(8):
                    off = r * 128 + k * 16
                    lane = ib * EBLK + off + _iota16()
                    s16 = cur_s[pl.ds(off, 16)]
                    d16 = cur_d[pl.ds(off, 16)]
                    loc = s16 - half_lo
                    m = (lane < 20000) & (loc >= 0) & (loc < HALF)
                    nrm = plsc.load_gather(dinv_t, [s16]) * plsc.load_gather(dinv_t, [d16])
                    b16 = plsc.load_gather(batch_t, [d16])
                    locc = jnp.where(m, loc, s16 & 4095)
                    cur_i[r, pl.ds(k * 16, 16)] = locc * N_GRAPHS + b16
                    cur_v[r, pl.ds(k * 16, 16)] = jnp.where(m, nrm, 0.0)

            _fire_scatter(ROWS, cur_i, cur_v, acc, cur_sem)

    _drain_scatter(ROWS, idx_b, val_b, acc, sem_s)
    _drain_scatter(ROWS, idx_b2, val_b2, acc, sem_s2)

    nbase = half_lo + s * 313

    @pl.loop(0, 3)
    def _(r):
        for k in range(8):
            off = r * 128 + k * 16
            lane = off + _iota16()
            nv = nbase + lane
            loc = nv - half_lo
            m = (lane < 313) & (loc < HALF)
            dv = plsc.load_gather(dinv_t, [nv])
            b16 = plsc.load_gather(batch_t, [nv])
            locc = jnp.where(m, loc, nv & 4095)
            idx_b[r, pl.ds(k * 16, 16)] = locc * N_GRAPHS + b16
            val_b[r, pl.ds(k * 16, 16)] = jnp.where(m, dv * dv, 0.0)

    _fire_scatter(3, idx_b, val_b, acc, sem_s)
    _drain_scatter(3, idx_b, val_b, acc, sem_s)
    plsc.subcore_barrier()

    out_w = SFLAT // 16
    _copy_out_async(bounce0, bounce1, acc, out_hbm, out_w, s * out_w, sc * SFLAT + s * out_w, 2048, sem_o)


NBLK_B = 6


@functools.partial(
    pl.kernel,
    out_type=jax.ShapeDtypeStruct((2 * CFLAT,), _f32),
    mesh=_mesh,
    compiler_params=_sc_params,
    scratch_types=[
        pltpu.VMEM((NP,), _f32),
        pltpu.VMEM((NP,), _i32),
        pltpu.VMEM((EBLK,), _i32),
        pltpu.VMEM((EBLK,), _i32),
        pltpu.VMEM((EBLK,), _i32),
        pltpu.VMEM((EBLK,), _i32),
        pltpu.VMEM((ROWS, 128), _i32),
        pltpu.VMEM((ROWS, 128), _f32),
        pltpu.VMEM((ROWS, 128), _i32),
        pltpu.VMEM((ROWS, 128), _f32),
        pltpu.VMEM((2048,), _f32),
        pltpu.VMEM((2048,), _f32),
        pltpu.VMEM_SHARED((CFLAT,), _f32),
        pltpu.SemaphoreType.DMA,
        pltpu.SemaphoreType.DMA,
        pltpu.SemaphoreType.DMA,
        pltpu.SemaphoreType.DMA,
        pltpu.SemaphoreType.DMA,
    ],
)
def _sc_cmat(ei_hbm, dinv_hbm, cls_hbm, out_hbm,
             dinv_t, cls_t, src_t0, src_t1, dst_t0, dst_t1, idx_b, val_b,
             idx_b2, val_b2, bounce0, bounce1, acc,
             sem_z, sem_e, sem_s, sem_s2, sem_o):
    sc = lax.axis_index("c")
    s = lax.axis_index("s")
    w = sc * 16 + s
    ebase = w * 10000
    nbase = w * 313

    _fill_zbuf(bounce0, 2048)
    _fire_zero(bounce0, acc, s * (CFLAT // 16), CFLAT // 16, 2048, sem_z)
    pltpu.sync_copy(dinv_hbm, dinv_t)
    pltpu.sync_copy(cls_hbm, cls_t)
    pltpu.async_copy(ei_hbm.at[pl.ds(ebase, EBLK)], src_t0, sem_e)
    pltpu.async_copy(ei_hbm.at[pl.ds(EP + ebase, EBLK)], dst_t0, sem_e)
    _drain_zero(bounce0, acc, s * (CFLAT // 16), CFLAT // 16, 2048, sem_z)
    plsc.subcore_barrier()

    @pl.loop(0, NBLK_B // 2)
    def _(jj):
        for half in range(2):
            ib = jj * 2 + half
            cur_s = src_t0 if half == 0 else src_t1
            cur_d = dst_t0 if half == 0 else dst_t1
            nxt_s = src_t1 if half == 0 else src_t0
            nxt_d = dst_t1 if half == 0 else dst_t0
            cur_i = idx_b if half == 0 else idx_b2
            cur_v = val_b if half == 0 else val_b2
            cur_sem = sem_s if half == 0 else sem_s2
            pltpu.make_async_copy(ei_hbm.at[pl.ds(ebase + ib * EBLK, EBLK)], cur_s, sem_e).wait()
            pltpu.make_async_copy(ei_hbm.at[pl.ds(EP + ebase + ib * EBLK, EBLK)], cur_d, sem_e).wait()

            @pl.when(ib + 1 < NBLK_B)
            def _(ib=ib, nxt_s=nxt_s, nxt_d=nxt_d):
                pltpu.async_copy(ei_hbm.at[pl.ds(ebase + (ib + 1) * EBLK, EBLK)], nxt_s, sem_e)
                pltpu.async_copy(ei_hbm.at[pl.ds(EP + ebase + (ib + 1) * EBLK, EBLK)], nxt_d, sem_e)

            @pl.when(ib >= 2)
            def _(cur_i=cur_i, cur_v=cur_v, cur_sem=cur_sem):
                _drain_scatter(ROWS, cur_i, cur_v, acc, cur_sem)

            @pl.loop(0, ROWS)
            def _(r, ib=ib, cur_s=cur_s, cur_d=cur_d, cur_i=cur_i, cur_v=cur_v):
                for k in range(8):
                    off = r * 128 + k * 16
                    lane = ib * EBLK + off + _iota16()
                    m = lane < 10000
                    s16 = cur_s[pl.ds(off, 16)]
                    d16 = cur_d[pl.ds(off, 16)]
                    nrm = plsc.load_gather(dinv_t, [s16]) * plsc.load_gather(dinv_t, [d16])
                    c16 = plsc.load_gather(cls_t, [s16])
                    pnd = jnp.where(d16 >= HALF, d16 + 120, d16)
                    cur_i[r, pl.ds(k * 16, 16)] = pnd * 128 + c16
                    cur_v[r, pl.ds(k * 16, 16)] = jnp.where(m, nrm, 0.0)

            _fire_scatter(ROWS, cur_i, cur_v, acc, cur_sem)

    _drain_scatter(ROWS, idx_b, val_b, acc, sem_s)
    _drain_scatter(ROWS, idx_b2, val_b2, acc, sem_s2)

    @pl.loop(0, 3)
    def _(r):
        for k in range(8):
            off = r * 128 + k * 16
            lane = off + _iota16()
            nv = nbase + lane
            m = (lane < 313) & (nv < N_NODES)
            dv = plsc.load_gather(dinv_t, [nv])
            c16 = plsc.load_gather(cls_t, [nv])
            pnn = jnp.where(nv >= HALF, nv + 120, nv)
            idx_b[r, pl.ds(k * 16, 16)] = pnn * 128 + c16
            val_b[r, pl.ds(k * 16, 16)] = jnp.where(m, dv * dv, 0.0)

    _fire_scatter(3, idx_b, val_b, acc, sem_s)
    _drain_scatter(3, idx_b, val_b, acc, sem_s)
    plsc.subcore_barrier()

    out_w = CFLAT // 16
    _copy_out_async(bounce0, bounce1, acc, out_hbm, out_w, s * out_w, sc * CFLAT + s * out_w, 2048, sem_o)


KBLK = 256
_bf16 = jnp.bfloat16


def _tc_body(c0_ref, c1_ref, emb_ref, w1_ref, b1_ref, w2_ref, st_ref, cnt_ref,
             b2_ref, out_ref, t1_ref, w2b_ref):
    @pl.when((pl.program_id(0) == 0) & (pl.program_id(1) == 0))
    def _():
        t1_ref[...] = jnp.dot(emb_ref[...], w1_ref[...],
                              preferred_element_type=_f32).astype(_bf16)
        w2b_ref[...] = w2_ref[...].astype(_bf16)
        out_ref[...] = cnt_ref[...] * b2_ref[...]

    a = (c0_ref[...] + c1_ref[...]).astype(_bf16)
    h = jnp.dot(a, t1_ref[...], preferred_element_type=_f32) + b1_ref[...]
    h = jnp.maximum(h, 0.0).astype(_bf16)
    hw = jnp.dot(h, w2b_ref[...], preferred_element_type=_f32).astype(_bf16)
    st = st_ref[...].astype(_bf16)
    out_ref[...] += lax.dot_general(
        st, hw, (((0,), (0,)), ((), ())), preferred_element_type=_f32)


def _tc_dense(c2, embed_p, w1, b1r, w2, st2, cntc, b2r):
    nk = HP // KBLK
    return pl.pallas_call(
        _tc_body,
        grid=(2, nk),
        in_specs=[
            pl.BlockSpec((KBLK, 128), lambda s, k: (s * (HP // KBLK) + k, 0)),
            pl.BlockSpec((KBLK, 128), lambda s, k: (NP // KBLK + s * (HP // KBLK) + k, 0)),
            pl.BlockSpec((128, 128), lambda s, k: (0, 0)),
            pl.BlockSpec((128, 128), lambda s, k: (0, 0)),
            pl.BlockSpec((1, 128), lambda s, k: (0, 0)),
            pl.BlockSpec((128, 128), lambda s, k: (0, 0)),
            pl.BlockSpec((KBLK, N_GRAPHS), lambda s, k: (s * (HP // KBLK) + k, 0)),
            pl.BlockSpec((N_GRAPHS, 1), lambda s, k: (0, 0)),
            pl.BlockSpec((1, 128), lambda s, k: (0, 0)),
        ],
        out_specs=pl.BlockSpec((N_GRAPHS, 128), lambda s, k: (0, 0)),
        out_shape=jax.ShapeDtypeStruct((N_GRAPHS, 128), _f32),
        scratch_shapes=[pltpu.VMEM((128, 128), _bf16), pltpu.VMEM((128, 128), _bf16)],
    )(c2, c2, embed_p, w1, b1r, w2, st2, cntc, b2r)


def kernel(x, edge_index, batch, embed_atom, W1, b1, W2, b2):
    cls_p = jnp.pad(x[:, 0].astype(_i32), (0, NP - N_NODES))
    batch_p = jnp.pad(batch.astype(_i32), (0, NP - N_NODES))
    ei_p = jnp.pad(edge_index.astype(_i32), ((0, 0), (0, EP - N_EDGES))).reshape(2 * EP)
    embed_p = jnp.pad(embed_atom.astype(_f32), ((0, 128 - embed_atom.shape[0]), (0, 0)))
    b1r = b1.reshape(1, HID).astype(_f32)
    b2r = b2.reshape(1, HID).astype(_f32)

    cnt, dinv, smat = _sc_smat(ei_p, batch_p)
    cmat = _sc_cmat(ei_p, dinv, cls_p)

    c2 = cmat.reshape(2 * NP, 128)
    st2 = smat.reshape(2 * HP, N_GRAPHS)
    return _tc_dense(c2, embed_p, W1.astype(_f32), b1r,
                     W2.astype(_f32), st2, cnt.reshape(N_GRAPHS, 1), b2r)
